# Initial kernel scaffold; baseline (speedup 1.0000x reference)
import jax
import jax.numpy as jnp
from jax.experimental import pallas as pl


def kernel(x, batch_num, level_11_edge_index, level_22_edge_index, level_21_seg, level_32_seg, level_13_indicator, params):
    raise NotImplementedError("write your pallas kernel here")


# speedup vs baseline: 7.4566x; 7.4566x over previous
import functools
import math

import jax
import jax.numpy as jnp
from jax import lax
from jax.experimental import pallas as pl
from jax.experimental.pallas import tpu as pltpu
from jax.experimental.pallas import tpu_sc as plsc

L1N = 10000; L2N = 1000; L3N = 50; L1E = 160000; L2E = 16000; HID = 128; BN = 4

NP1 = 10240
NP2 = 1024
NP3 = 1024
EP1 = 163840
EP2 = 16384
SP1 = 12288
NW = 32
NSUB = 16
EPS = 1e-5
_STOP_AT = 99

_mesh = None
_SC_PARAMS = pltpu.CompilerParams(needs_layout_passes=False)


def _get_mesh():
    global _mesh
    if _mesh is None:
        _mesh = plsc.VectorSubcoreMesh(core_axis_name="c", subcore_axis_name="s",
                                       num_cores=2, num_subcores=NSUB)
    return _mesh


def _wid():
    return lax.axis_index("s") * 2 + lax.axis_index("c")


def _zero_vmem(ref, n):
    z = jnp.zeros((16,), jnp.float32)

    def body(i, _):
        ref[pl.ds(i * 16, 16)] = z
        return 0

    lax.fori_loop(0, n // 16, body, 0)


def _zero_vmem2d(ref, rows):
    z = jnp.zeros((16,), jnp.float32)

    def body(i, _):
        for j in range(8):
            ref[i, pl.ds(j * 16, 16)] = z
        return 0

    lax.fori_loop(0, rows, body, 0)


def _blocks(ep):
    ew = ep // NW
    if ew >= 128:
        assert ew % 128 == 0
        return ew, 128, ew // 128
    assert ew % 16 == 0
    return ew, ew, 1


@functools.lru_cache(maxsize=None)
def _make_sc_deg(ep, t):
    ew, bs, nb = _blocks(ep)
    zl = t // NSUB

    @functools.partial(
        pl.kernel,
        out_type=jax.ShapeDtypeStruct((2 * t,), jnp.float32),
        mesh=_get_mesh(),
        compiler_params=_SC_PARAMS,
        scratch_types=[
            pltpu.VMEM((bs,), jnp.int32),
            pltpu.VMEM((bs,), jnp.float32),
            pltpu.VMEM((zl,), jnp.float32),
            pltpu.VMEM_SHARED((t,), jnp.float32),
        ],
    )
    def k(dst_hbm, out_hbm, idx_v, ones_v, zbuf, acc_sh):
        c = lax.axis_index("c")
        s = lax.axis_index("s")
        wid = _wid()
        _zero_vmem(zbuf, zl)
        _zero_vmem(ones_v, bs)
        ones = jnp.ones((16,), jnp.float32)

        def fill(i, _):
            ones_v[pl.ds(i * 16, 16)] = ones
            return 0

        lax.fori_loop(0, bs // 16, fill, 0)
        pltpu.sync_copy(zbuf, acc_sh.at[pl.ds(s * zl, zl)])
        plsc.subcore_barrier()

        def blk(i, _):
            base = wid * ew + i * bs
            pltpu.sync_copy(dst_hbm.at[pl.ds(base, bs)], idx_v)
            pltpu.sync_copy(ones_v, acc_sh.at[idx_v], add=True)
            return 0

        lax.fori_loop(0, nb, blk, 0)
        plsc.subcore_barrier()
        pltpu.sync_copy(acc_sh.at[pl.ds(s * zl, zl)],
                        out_hbm.at[pl.ds(c * t + s * zl, zl)])

    return k


@functools.lru_cache(maxsize=None)
def _make_sc_edgew(ep, t):
    ew, bs, nb = _blocks(ep)

    @functools.partial(
        pl.kernel,
        out_type=jax.ShapeDtypeStruct((ep,), jnp.float32),
        mesh=_get_mesh(),
        compiler_params=_SC_PARAMS,
        scratch_types=[
            pltpu.VMEM((t,), jnp.float32),
            pltpu.VMEM((bs,), jnp.int32),
            pltpu.VMEM((bs,), jnp.int32),
            pltpu.VMEM((bs,), jnp.float32),
        ],
    )
    def k(dinv_hbm, src_hbm, dst_hbm, out_hbm, dtab, sv, dv, ov):
        wid = _wid()
        pltpu.sync_copy(dinv_hbm, dtab)

        def blk(i, _):
            base = wid * ew + i * bs

            pltpu.sync_copy(src_hbm.at[pl.ds(base, bs)], sv)
            pltpu.sync_copy(dst_hbm.at[pl.ds(base, bs)], dv)

            def vec(j, _):
                svj = sv[pl.ds(j * 16, 16)]
                dvj = dv[pl.ds(j * 16, 16)]
                a = plsc.load_gather(dtab, [svj])
                b = plsc.load_gather(dtab, [dvj])
                ov[pl.ds(j * 16, 16)] = a * b
                return 0

            lax.fori_loop(0, bs // 16, vec, 0)
            pltpu.sync_copy(ov, out_hbm.at[pl.ds(base, bs)])
            return 0

        lax.fori_loop(0, nb, blk, 0)

    return k


@functools.lru_cache(maxsize=None)
def _make_sc_gcn1(ep, n, t):
    ew, bs, nb = _blocks(ep)
    zl = 4 * t // NSUB

    @functools.partial(
        pl.kernel,
        out_type=jax.ShapeDtypeStruct((2 * 4 * t,), jnp.float32),
        mesh=_get_mesh(),
        compiler_params=_SC_PARAMS,
        scratch_types=[
            pltpu.VMEM((4 * n,), jnp.float32),
            pltpu.VMEM((bs,), jnp.int32),
            pltpu.VMEM((bs,), jnp.int32),
            pltpu.VMEM((bs,), jnp.float32),
            pltpu.VMEM((bs,), jnp.int32),
            pltpu.VMEM((bs,), jnp.float32),
            pltpu.VMEM((zl,), jnp.float32),
            pltpu.VMEM_SHARED((4 * t,), jnp.float32),
        ],
    )
    def k(x_hbm, src_hbm, dst_hbm, w_hbm, out_hbm,
          xtab, sv, dv, wv, iv, vv, zbuf, acc_sh):
        c = lax.axis_index("c")
        s = lax.axis_index("s")
        wid = _wid()
        pltpu.sync_copy(x_hbm, xtab)
        _zero_vmem(zbuf, zl)
        pltpu.sync_copy(zbuf, acc_sh.at[pl.ds(s * zl, zl)])
        plsc.subcore_barrier()

        def blk(i, _):
            base = wid * ew + i * bs
            pltpu.sync_copy(src_hbm.at[pl.ds(base, bs)], sv)
            pltpu.sync_copy(dst_hbm.at[pl.ds(base, bs)], dv)
            pltpu.sync_copy(w_hbm.at[pl.ds(base, bs)], wv)
            for b in range(4):
                def vec(j, _):
                    svj = sv[pl.ds(j * 16, 16)] + (b * n)
                    xg = plsc.load_gather(xtab, [svj])
                    vv[pl.ds(j * 16, 16)] = xg * wv[pl.ds(j * 16, 16)]
                    iv[pl.ds(j * 16, 16)] = dv[pl.ds(j * 16, 16)] + (b * t)
                    return 0

                lax.fori_loop(0, bs // 16, vec, 0)
                pltpu.sync_copy(vv, acc_sh.at[iv], add=True)
            return 0

        lax.fori_loop(0, nb, blk, 0)
        plsc.subcore_barrier()
        pltpu.sync_copy(acc_sh.at[pl.ds(s * zl, zl)],
                        out_hbm.at[pl.ds(c * 4 * t + s * zl, zl)])

    return k


@functools.lru_cache(maxsize=None)
def _make_sc_segden(ep, t):
    ew, bs, nb = _blocks(ep)
    zl = 4 * t // NSUB
    small = zl < 256
    if small:
        zl = 4 * t

    @functools.partial(
        pl.kernel,
        out_type=jax.ShapeDtypeStruct((2 * 4 * t,), jnp.float32),
        mesh=_get_mesh(),
        compiler_params=_SC_PARAMS,
        scratch_types=[
            pltpu.VMEM((bs,), jnp.int32),
            pltpu.VMEM((bs,), jnp.int32),
            pltpu.VMEM((bs,), jnp.float32),
            pltpu.VMEM((zl,), jnp.float32),
            pltpu.VMEM_SHARED((4 * t,), jnp.float32),
        ],
    )
    def k(vals_hbm, dst_hbm, out_hbm, dv, iv, vv, zbuf, acc_sh):
        c = lax.axis_index("c")
        s = lax.axis_index("s")
        wid = _wid()
        _zero_vmem(zbuf, zl)
        if small:
            @pl.when(s == 0)
            def _():
                pltpu.sync_copy(zbuf, acc_sh)
        else:
            pltpu.sync_copy(zbuf, acc_sh.at[pl.ds(s * zl, zl)])
        plsc.subcore_barrier()

        def blk(i, _):
            base = wid * ew + i * bs
            pltpu.sync_copy(dst_hbm.at[pl.ds(base, bs)], dv)
            for b in range(4):
                pltpu.sync_copy(vals_hbm.at[b, pl.ds(base, bs)], vv)

                def vec(j, _):
                    iv[pl.ds(j * 16, 16)] = dv[pl.ds(j * 16, 16)] + (b * t)
                    return 0

                lax.fori_loop(0, bs // 16, vec, 0)
                pltpu.sync_copy(vv, acc_sh.at[iv], add=True)
            return 0

        lax.fori_loop(0, nb, blk, 0)
        plsc.subcore_barrier()
        if small:
            @pl.when(s == 0)
            def _():
                pltpu.sync_copy(acc_sh, out_hbm.at[pl.ds(c * 4 * t, 4 * t)])
        else:
            pltpu.sync_copy(acc_sh.at[pl.ds(s * zl, zl)],
                            out_hbm.at[pl.ds(c * 4 * t + s * zl, zl)])

    return k


@functools.lru_cache(maxsize=None)
def _make_sc_attdiv(n, t):
    ew, bs, nb = _blocks(n)

    @functools.partial(
        pl.kernel,
        out_type=jax.ShapeDtypeStruct((4, n), jnp.float32),
        mesh=_get_mesh(),
        compiler_params=_SC_PARAMS,
        scratch_types=[
            pltpu.VMEM((4 * t,), jnp.float32),
            pltpu.VMEM((4 * t,), jnp.float32),
            pltpu.VMEM((4 * t,), jnp.float32),
            pltpu.VMEM((bs,), jnp.int32),
            pltpu.VMEM((bs,), jnp.float32),
            pltpu.VMEM((bs,), jnp.float32),
        ],
    )
    def k(num_hbm, d0_hbm, d1_hbm, d2_hbm, seg_hbm, out_hbm,
          t0, t1, t2, gv, nv, ov):
        wid = _wid()
        pltpu.sync_copy(d0_hbm, t0)
        pltpu.sync_copy(d1_hbm, t1)
        pltpu.sync_copy(d2_hbm, t2)

        def blk(i, _):
            base = wid * ew + i * bs
            pltpu.sync_copy(seg_hbm.at[pl.ds(base, bs)], gv)
            for b in range(4):
                pltpu.sync_copy(num_hbm.at[b, pl.ds(base, bs)], nv)

                def vec(j, _):
                    sl = pl.ds(j * 16, 16)
                    idx = gv[sl] + (b * t)
                    den = (plsc.load_gather(t0, [idx])
                           + plsc.load_gather(t1, [idx])
                           + plsc.load_gather(t2, [idx]) + 1e-16)
                    ov[sl] = nv[sl] / den
                    return 0

                lax.fori_loop(0, bs // 16, vec, 0)
                pltpu.sync_copy(ov, out_hbm.at[b, pl.ds(base, bs)])
            return 0

        lax.fori_loop(0, nb, blk, 0)

    return k


@functools.lru_cache(maxsize=None)
def _make_sc_gatex(ep, t):
    ew, bs, nb = _blocks(ep)

    @functools.partial(
        pl.kernel,
        out_type=jax.ShapeDtypeStruct((4, ep), jnp.float32),
        mesh=_get_mesh(),
        compiler_params=_SC_PARAMS,
        scratch_types=[
            pltpu.VMEM((4 * t,), jnp.float32),
            pltpu.VMEM((4 * t,), jnp.float32),
            pltpu.VMEM((64,), jnp.float32),
            pltpu.VMEM((bs,), jnp.int32),
            pltpu.VMEM((bs,), jnp.int32),
            pltpu.VMEM((bs,), jnp.float32),
        ],
    )
    def k(hs_hbm, hd_hbm, c_hbm, src_hbm, dst_hbm, out_hbm,
          ts, td, tc, sv, dv, ov):
        wid = _wid()
        pltpu.sync_copy(hs_hbm, ts)
        pltpu.sync_copy(hd_hbm, td)
        pltpu.sync_copy(c_hbm, tc)

        def blk(i, _):
            base = wid * ew + i * bs
            pltpu.sync_copy(src_hbm.at[pl.ds(base, bs)], sv)
            pltpu.sync_copy(dst_hbm.at[pl.ds(base, bs)], dv)
            for b in range(4):
                cb = tc[pl.ds(b * 16, 16)]

                def vec(j, _):
                    sl = pl.ds(j * 16, 16)
                    a = plsc.load_gather(ts, [sv[sl] + (b * t)])
                    d = plsc.load_gather(td, [dv[sl] + (b * t)])
                    e = a + d
                    e = jnp.where(e >= 0.0, e, 0.2 * e)
                    ov[sl] = jnp.exp(e - cb)
                    return 0

                lax.fori_loop(0, bs // 16, vec, 0)
                pltpu.sync_copy(ov, out_hbm.at[b, pl.ds(base, bs)])
            return 0

        lax.fori_loop(0, nb, blk, 0)

    return k


@functools.lru_cache(maxsize=None)
def _make_sc_rows(ep, n, mp):
    ew, bs, nb = _blocks(ep)
    rpt = mp // NSUB
    ztail = rpt % 128
    nzfull = rpt // 128

    @functools.partial(
        pl.kernel,
        out_type=jax.ShapeDtypeStruct((2 * mp, 128), jnp.float32),
        mesh=_get_mesh(),
        compiler_params=_SC_PARAMS,
        scratch_types=[
            pltpu.VMEM((bs,), jnp.int32),
            pltpu.VMEM((bs,), jnp.int32),
            pltpu.VMEM((bs,), jnp.float32),
            pltpu.VMEM((bs, 128), jnp.float32),
            pltpu.VMEM_SHARED((mp, 128), jnp.float32),
            pltpu.SemaphoreType.DMA,
        ],
    )
    def k(table_hbm, src_hbm, dst_hbm, w_hbm, out_hbm,
          sv, dv, wv, rows, acc_sh, sem):
        c = lax.axis_index("c")
        s = lax.axis_index("s")
        wid = _wid()

        _zero_vmem2d(rows, min(bs, 128))
        zb = min(bs, 128)
        nf = rpt // zb
        zt = rpt % zb
        for tzi in range(nf):
            pltpu.sync_copy(rows.at[pl.ds(0, zb)],
                            acc_sh.at[pl.ds(s * rpt + tzi * zb, zb)])
        if zt:
            pltpu.sync_copy(rows.at[pl.ds(0, zt)],
                            acc_sh.at[pl.ds(s * rpt + nf * zb, zt)])
        plsc.subcore_barrier()

        def blk(i, _):
            base = wid * ew + i * bs
            pltpu.sync_copy(src_hbm.at[pl.ds(base, bs)], sv)
            pltpu.sync_copy(dst_hbm.at[pl.ds(base, bs)], dv)
            pltpu.sync_copy(w_hbm.at[pl.ds(base, bs)], wv)
            pltpu.async_copy(table_hbm.at[sv], rows, sem).wait()

            def scale(r, _):
                wr = plsc.load_gather(wv, [jnp.full((16,), r, jnp.int32)])
                for j in range(8):
                    sl = pl.ds(j * 16, 16)
                    rows[r, sl] = rows[r, sl] * wr
                return 0

            lax.fori_loop(0, bs, scale, 0)
            pltpu.sync_copy(rows, acc_sh.at[dv], add=True)
            return 0

        lax.fori_loop(0, nb, blk, 0)
        plsc.subcore_barrier()

        base_o = c * mp + s * rpt
        nco = (rpt + 127) // 128
        for t2 in range(nco):
            sz = min(128, rpt - t2 * 128)
            pltpu.sync_copy(acc_sh.at[pl.ds(s * rpt + t2 * 128, sz)],
                            out_hbm.at[pl.ds(base_o + t2 * 128, sz)])

    return k


def _tc_dinv(degp):
    def body(d_ref, o_ref):
        d = d_ref[0] + d_ref[1] + 1.0
        o_ref[...] = lax.rsqrt(d.reshape(80, 128))

    out = pl.pallas_call(
        body,
        out_shape=jax.ShapeDtypeStruct((80, 128), jnp.float32),
    )(degp.reshape(2, 80, 128))
    return out.reshape(NP1)


def _tc_gcn1ln(alphap, x, dinv, w1, g, b):
    blk = 512
    nj = NP1 // blk

    def body(a_ref, x_ref, di_ref, w_ref, g_ref, b_ref, o_ref):
        alpha = (a_ref[0] + a_ref[1]
                 + di_ref[0] * di_ref[0] * x_ref[0])
        w = w_ref[0]
        mw = jnp.mean(w)
        vw = jnp.mean((w - mw) ** 2)
        wc = w - mw
        denom = lax.rsqrt(alpha * alpha * vw + EPS)
        o_ref[0] = ((alpha * denom)[:, None] * wc[None, :] * g_ref[0][None, :]
                    + b_ref[0][None, :])

    grid = (BN, nj)
    return pl.pallas_call(
        body,
        grid=grid,
        in_specs=[
            pl.BlockSpec((2, blk), lambda b2, j: (0, b2 * nj + j)),
            pl.BlockSpec((1, blk), lambda b2, j: (0, b2 * nj + j)),
            pl.BlockSpec((1, blk), lambda b2, j: (0, j)),
            pl.BlockSpec((1, HID), lambda b2, j: (0, 0)),
            pl.BlockSpec((1, HID), lambda b2, j: (0, 0)),
            pl.BlockSpec((1, HID), lambda b2, j: (0, 0)),
        ],
        out_specs=pl.BlockSpec((1, blk, HID), lambda b2, j: (b2, j, 0)),
        out_shape=jax.ShapeDtypeStruct((BN, NP1, HID), jnp.float32),
    )(alphap.reshape(2, BN * NP1), x.reshape(1, BN * NP1),
      dinv.reshape(1, NP1), w1.reshape(1, HID),
      g.reshape(1, HID), b.reshape(1, HID))


def _tc_matmul(h, w):
    m, kdim = h.shape
    kout = w.shape[1]
    blk = 512

    def body(h_ref, w_ref, o_ref):
        o_ref[...] = jnp.dot(h_ref[...], w_ref[...],
                             preferred_element_type=jnp.float32)

    return pl.pallas_call(
        body,
        grid=(m // blk,),
        in_specs=[
            pl.BlockSpec((blk, kdim), lambda i: (i, 0)),
            pl.BlockSpec((kdim, kout), lambda i: (0, 0)),
        ],
        out_specs=pl.BlockSpec((blk, kout), lambda i: (i, 0)),
        out_shape=jax.ShapeDtypeStruct((m, kout), jnp.float32),
    )(h, w)


def _tc_gcn2post(aggp, h2, h1, dinv, b2, g, b):
    blk = 512

    def body(a_ref, h2_ref, h1_ref, di_ref, b2_ref, g_ref, b_ref, o_ref):
        pre = (a_ref[0, 0] + a_ref[1, 0]
               + (di_ref[0] * di_ref[0])[:, None] * h2_ref[0]
               + b2_ref[0][None, :])
        m = jnp.mean(pre, axis=-1, keepdims=True)
        v = jnp.mean((pre - m) ** 2, axis=-1, keepdims=True)
        o_ref[0] = h1_ref[0] + ((pre - m) * lax.rsqrt(v + EPS)
                                * g_ref[0][None, :] + b_ref[0][None, :])

    grid = (BN, NP1 // blk)
    return pl.pallas_call(
        body,
        grid=grid,
        in_specs=[
            pl.BlockSpec((2, 1, blk, HID), lambda b3, j: (0, b3, j, 0)),
            pl.BlockSpec((1, blk, HID), lambda b3, j: (b3, j, 0)),
            pl.BlockSpec((1, blk, HID), lambda b3, j: (b3, j, 0)),
            pl.BlockSpec((1, blk), lambda b3, j: (0, j)),
            pl.BlockSpec((1, HID), lambda b3, j: (0, 0)),
            pl.BlockSpec((1, HID), lambda b3, j: (0, 0)),
            pl.BlockSpec((1, HID), lambda b3, j: (0, 0)),
        ],
        out_specs=pl.BlockSpec((1, blk, HID), lambda b3, j: (b3, j, 0)),
        out_shape=jax.ShapeDtypeStruct((BN, NP1, HID), jnp.float32),
    )(aggp, h2, h1, dinv.reshape(1, NP1), b2.reshape(1, HID),
      g.reshape(1, HID), b.reshape(1, HID))


def _tc_s2t_scores(hflat, p):
    m = hflat.shape[0]
    blk = 512
    f1w, f1b = p['fc1_W'], p['fc1_b']
    lng, lnb = p['ln_g'], p['ln_b']
    f2w = p['fc2_W'][:, 0]
    f2b = p['fc2_b'][0]

    def body(h_ref, w1_ref, b1_ref, g_ref, b_ref, w2_ref, b2_ref, o_ref):
        z = jnp.dot(h_ref[...], w1_ref[...],
                    preferred_element_type=jnp.float32) + b1_ref[0][None, :]
        mu = jnp.mean(z, axis=-1, keepdims=True)
        v = jnp.mean((z - mu) ** 2, axis=-1, keepdims=True)
        t = jnp.tanh((z - mu) * lax.rsqrt(v + EPS) * g_ref[0][None, :]
                     + b_ref[0][None, :])
        o_ref[0] = jnp.sum(t * w2_ref[0][None, :], axis=-1) + b2_ref[0, 0]

    return pl.pallas_call(
        body,
        grid=(m // blk,),
        in_specs=[
            pl.BlockSpec((blk, HID), lambda i: (i, 0)),
            pl.BlockSpec((HID, 4 * HID), lambda i: (0, 0)),
            pl.BlockSpec((1, 4 * HID), lambda i: (0, 0)),
            pl.BlockSpec((1, 4 * HID), lambda i: (0, 0)),
            pl.BlockSpec((1, 4 * HID), lambda i: (0, 0)),
            pl.BlockSpec((1, 4 * HID), lambda i: (0, 0)),
            pl.BlockSpec((1, 1), lambda i: (0, 0)),
        ],
        out_specs=pl.BlockSpec((1, blk), lambda i: (0, i)),
        out_shape=jax.ShapeDtypeStruct((1, m), jnp.float32),
    )(hflat, f1w, f1b.reshape(1, -1), lng.reshape(1, -1),
      lnb.reshape(1, -1), f2w.reshape(1, -1),
      p['fc2_b'].reshape(1, 1)).reshape(m)


def _tc_sexp(s, npad, nvalid):
    def body(s_ref, o_ref):
        row = s_ref[0]
        iot = lax.broadcasted_iota(jnp.int32, (npad,), 0)
        msk = iot < nvalid
        sm = jnp.where(msk, row, -1e30)
        cmax = jnp.max(sm)
        o_ref[0] = jnp.where(msk, jnp.exp(row - cmax), 0.0)

    return pl.pallas_call(
        body,
        grid=(BN,),
        in_specs=[pl.BlockSpec((1, npad), lambda b: (0, b))],
        out_specs=pl.BlockSpec((1, npad), lambda b: (0, b)),
        out_shape=jax.ShapeDtypeStruct((1, BN * npad), jnp.float32),
    )(s.reshape(1, BN * npad)).reshape(BN, npad)


def _tc_gatpre(lv2p, gw, a_s, a_d, nvalid):
    def body(l_ref, w_ref, as_ref, ad_ref, hw_ref, hs_ref, hd_ref,
             c_ref, es_ref):
        lv2 = l_ref[0, 0] + l_ref[1, 0]
        hw = jnp.dot(lv2, w_ref[...], preferred_element_type=jnp.float32)
        hs = jnp.sum(hw * as_ref[0][None, :], axis=-1)
        hd = jnp.sum(hw * ad_ref[0][None, :], axis=-1)
        iot = lax.broadcasted_iota(jnp.int32, (NP2,), 0)
        msk = iot < nvalid
        hs = jnp.where(msk, hs, 0.0)
        hd = jnp.where(msk, hd, 0.0)
        cb = jnp.maximum(jnp.max(jnp.where(msk, hs, -1e30))
                         + jnp.max(jnp.where(msk, hd, -1e30)), 0.0)
        e = hs + hd
        e = jnp.where(e >= 0.0, e, 0.2 * e)
        hw_ref[0] = hw
        hs_ref[0] = hs
        hd_ref[0] = hd
        c_ref[0] = jnp.full((NP2,), cb, jnp.float32)
        es_ref[0] = jnp.where(msk, jnp.exp(e - cb), 0.0)

    outs = pl.pallas_call(
        body,
        grid=(BN,),
        in_specs=[
            pl.BlockSpec((2, 1, NP2, HID), lambda b: (0, b, 0, 0)),
            pl.BlockSpec((HID, HID), lambda b: (0, 0)),
            pl.BlockSpec((1, HID), lambda b: (0, 0)),
            pl.BlockSpec((1, HID), lambda b: (0, 0)),
        ],
        out_specs=[
            pl.BlockSpec((1, NP2, HID), lambda b: (b, 0, 0)),
            pl.BlockSpec((1, NP2), lambda b: (0, b)),
            pl.BlockSpec((1, NP2), lambda b: (0, b)),
            pl.BlockSpec((1, NP2), lambda b: (0, b)),
            pl.BlockSpec((1, NP2), lambda b: (0, b)),
        ],
        out_shape=[
            jax.ShapeDtypeStruct((BN, NP2, HID), jnp.float32),
            jax.ShapeDtypeStruct((1, BN * NP2), jnp.float32),
            jax.ShapeDtypeStruct((1, BN * NP2), jnp.float32),
            jax.ShapeDtypeStruct((1, BN * NP2), jnp.float32),
            jax.ShapeDtypeStruct((1, BN * NP2), jnp.float32),
        ],
    )(lv2p, gw, a_s.reshape(1, HID), a_d.reshape(1, HID))
    return outs


def _tc_gatpost(gaggp, hw, ex_self, d0, d1, gb, g, b):
    def body(a_ref, hw_ref, es_ref, d0_ref, d1_ref, gb_ref, g_ref, b_ref,
             o_ref):
        es = es_ref[0]
        den = d0_ref[0] + d1_ref[0] + es + 1e-16
        atts = es / den
        pre = (a_ref[0, 0] + a_ref[1, 0] + atts[:, None] * hw_ref[0]
               + gb_ref[0][None, :])
        m = jnp.mean(pre, axis=-1, keepdims=True)
        v = jnp.mean((pre - m) ** 2, axis=-1, keepdims=True)
        o_ref[0] = ((pre - m) * lax.rsqrt(v + EPS) * g_ref[0][None, :]
                    + b_ref[0][None, :])

    return pl.pallas_call(
        body,
        grid=(BN,),
        in_specs=[
            pl.BlockSpec((2, 1, NP2, HID), lambda b2: (0, b2, 0, 0)),
            pl.BlockSpec((1, NP2, HID), lambda b2: (b2, 0, 0)),
            pl.BlockSpec((1, NP2), lambda b2: (0, b2)),
            pl.BlockSpec((1, NP2), lambda b2: (0, b2)),
            pl.BlockSpec((1, NP2), lambda b2: (0, b2)),
            pl.BlockSpec((1, HID), lambda b2: (0, 0)),
            pl.BlockSpec((1, HID), lambda b2: (0, 0)),
            pl.BlockSpec((1, HID), lambda b2: (0, 0)),
        ],
        out_specs=pl.BlockSpec((1, NP2, HID), lambda b2: (b2, 0, 0)),
        out_shape=jax.ShapeDtypeStruct((BN, NP2, HID), jnp.float32),
    )(gaggp, hw, ex_self.reshape(1, BN * NP2), d0.reshape(1, BN * NP2),
      d1.reshape(1, BN * NP2), gb.reshape(1, HID),
      g.reshape(1, HID), b.reshape(1, HID))


def _tc_final(parts):
    def body(p_ref, o_ref):
        o_ref[0] = p_ref[0, 0] + p_ref[1, 0]

    return pl.pallas_call(
        body,
        grid=(BN,),
        in_specs=[pl.BlockSpec((2, 1, NP3, HID), lambda b: (0, b, 0, 0))],
        out_specs=pl.BlockSpec((1, NP3, HID), lambda b: (b, 0, 0)),
        out_shape=jax.ShapeDtypeStruct((BN, NP3, HID), jnp.float32),
    )(parts)


def kernel(x, batch_num, level_11_edge_index, level_22_edge_index,
           level_21_seg, level_32_seg, level_13_indicator, params):
    p = params
    B = x.shape[0]

    i32 = jnp.int32
    src1 = level_11_edge_index[0].astype(i32)
    dst1 = level_11_edge_index[1].astype(i32)
    pad1 = jnp.full((EP1 - L1E,), NP1 - 1, i32)
    src1p = jnp.concatenate([src1, pad1])
    dst1p = jnp.concatenate([dst1, pad1])

    src2 = level_22_edge_index[0].astype(i32)
    dst2 = level_22_edge_index[1].astype(i32)
    pad2 = jnp.full((EP2 - L2E,), NP2 - 1, i32)
    src2p = jnp.concatenate([src2, pad2])
    dst2p = jnp.concatenate([dst2, pad2])

    seg21 = level_21_seg.astype(i32)
    seg21p = jnp.concatenate([seg21, jnp.full((SP1 - L1N,), NP2 - 1, i32)])
    iota1p = jnp.minimum(jnp.arange(SP1, dtype=i32), NP1 - 1)

    seg32 = level_32_seg.astype(i32)
    seg32p = jnp.concatenate([seg32, jnp.full((NP2 - L2N,), NP3 - 1, i32)])
    iota2p = jnp.arange(NP2, dtype=i32)

    xp = jnp.pad(x, ((0, 0), (0, NP1 - L1N)))

    degp = _make_sc_deg(EP1, NP1)(dst1p)
    dinv = _tc_dinv(degp)
    w_e = _make_sc_edgew(EP1, NP1)(dinv, src1p, dst1p)

    alphap = _make_sc_gcn1(EP1, NP1, NP1)(
        xp.reshape(4 * NP1), src1p, dst1p, w_e)
    h1 = _tc_gcn1ln(alphap.reshape(2, BN, NP1), xp, dinv,
                    p['gcn1_W'][0], p['ln1_g'], p['ln1_b'])
    if _STOP_AT == 2:
        return jnp.zeros((BN, L3N, HID), jnp.float32) + 0.0 * h1[:, :L3N]

    h1f = h1.reshape(BN * NP1, HID)
    h2f = _tc_matmul(h1f, p['gcn2_W'])
    h2 = h2f.reshape(BN, NP1, HID)
    rows1 = _make_sc_rows(EP1, NP1, NP1)
    aggp = jnp.stack([rows1(h2[b], src1p, dst1p, w_e).reshape(2, NP1, HID)
                      for b in range(BN)], axis=1)
    h = _tc_gcn2post(aggp, h2, h1, dinv, p['gcn2_b'],
                     p['ln2_g'], p['ln2_b'])
    if _STOP_AT == 3:
        return jnp.zeros((BN, L3N, HID), jnp.float32) + 0.0 * h[:, :L3N]

    s12 = _tc_s2t_scores(h.reshape(BN * NP1, HID),
                         p['s2t12']).reshape(BN, NP1)
    ex12 = _tc_sexp(s12, NP1, L1N)
    ex12p = jnp.pad(ex12, ((0, 0), (0, SP1 - NP1)))
    den12 = _make_sc_segden(SP1, NP2)(ex12p, seg21p)
    d12 = den12.reshape(2, 4 * NP2)
    zeros2 = jnp.zeros((4 * NP2,), jnp.float32)
    att12 = _make_sc_attdiv(SP1, NP2)(ex12p, d12[0], d12[1], zeros2, seg21p)
    rows12 = _make_sc_rows(SP1, NP1, NP2)
    if _STOP_AT == 4:
        return (jnp.zeros((BN, L3N, HID), jnp.float32)
                + 0.0 * att12[:, :L3N, None])
    lv2p = jnp.stack([rows12(h[b], iota1p, seg21p,
                             att12[b]).reshape(2, NP2, HID)
                      for b in range(BN)], axis=1)

    hw, hs, hd, cb, ex_self = _tc_gatpre(lv2p, p['gat_W'],
                                         p['gat_as'], p['gat_ad'], L2N)
    cb64 = cb.reshape(BN, NP2)[:, :16].reshape(64)
    exg = _make_sc_gatex(EP2, NP2)(hs.reshape(4 * NP2), hd.reshape(4 * NP2),
                                   cb64, src2p, dst2p)
    deng = _make_sc_segden(EP2, NP2)(exg, dst2p)
    dg = deng.reshape(2, 4 * NP2)
    attg = _make_sc_attdiv(EP2, NP2)(exg, dg[0], dg[1],
                                     ex_self.reshape(4 * NP2), dst2p)
    rowsg = _make_sc_rows(EP2, NP2, NP2)
    gaggp = jnp.stack([rowsg(hw[b], src2p, dst2p, attg[b]).reshape(2, NP2, HID)
                       for b in range(BN)], axis=1)
    dgb = dg.reshape(2, BN, NP2)
    lv2f = _tc_gatpost(gaggp, hw, ex_self, dgb[0], dgb[1],
                       p['gat_b'], p['lng_g'], p['lng_b'])
    if _STOP_AT == 5:
        return jnp.zeros((BN, L3N, HID), jnp.float32) + 0.0 * lv2f[:, :L3N]

    s23 = _tc_s2t_scores(lv2f.reshape(BN * NP2, HID),
                         p['s2t23']).reshape(BN, NP2)
    ex23 = _tc_sexp(s23, NP2, L2N)
    den23 = _make_sc_segden(NP2, NP3)(ex23, seg32p)
    d23 = den23.reshape(2, 4 * NP3)
    zeros3 = jnp.zeros((4 * NP3,), jnp.float32)
    att23 = _make_sc_attdiv(NP2, NP3)(ex23, d23[0], d23[1], zeros3, seg32p)
    rows23 = _make_sc_rows(NP2, NP2, NP3)
    lv3p = jnp.stack([rows23(lv2f[b], iota2p, seg32p,
                             att23[b]).reshape(2, NP3, HID)
                      for b in range(BN)], axis=1)
    lv3 = _tc_final(lv3p)
    return lv3[:, :L3N, :]


# speedup vs baseline: 9.7443x; 1.3068x over previous
import functools
import math

import jax
import jax.numpy as jnp
from jax import lax
from jax.experimental import pallas as pl
from jax.experimental.pallas import tpu as pltpu
from jax.experimental.pallas import tpu_sc as plsc

L1N = 10000; L2N = 1000; L3N = 50; L1E = 160000; L2E = 16000; HID = 128; BN = 4

NP1 = 10240
NP2 = 1024
NP3 = 1024
EP1 = 163840
EP2 = 16384
SP1 = 12288
NW = 32
NSUB = 16
EPS = 1e-5
_STOP_AT = 99

_mesh = None
_SC_PARAMS = pltpu.CompilerParams(needs_layout_passes=False)


def _get_mesh():
    global _mesh
    if _mesh is None:
        _mesh = plsc.VectorSubcoreMesh(core_axis_name="c", subcore_axis_name="s",
                                       num_cores=2, num_subcores=NSUB)
    return _mesh


def _wid():
    return lax.axis_index("s") * 2 + lax.axis_index("c")


def _zero_vmem(ref, n):
    z = jnp.zeros((16,), jnp.float32)

    def body(i, _):
        ref[pl.ds(i * 16, 16)] = z
        return 0

    lax.fori_loop(0, n // 16, body, 0)


def _zero_vmem2d(ref, rows):
    z = jnp.zeros((16,), jnp.float32)

    def body(i, _):
        for j in range(8):
            ref[i, pl.ds(j * 16, 16)] = z
        return 0

    lax.fori_loop(0, rows, body, 0)


def _blocks(ep):
    ew = ep // NW
    if ew >= 128:
        assert ew % 128 == 0
        return ew, 128, ew // 128
    assert ew % 16 == 0
    return ew, ew, 1


@functools.lru_cache(maxsize=None)
def _make_sc_deg(ep, t):
    ew, bs, nb = _blocks(ep)
    zl = t // NSUB

    @functools.partial(
        pl.kernel,
        out_type=jax.ShapeDtypeStruct((2 * t,), jnp.float32),
        mesh=_get_mesh(),
        compiler_params=_SC_PARAMS,
        scratch_types=[
            pltpu.VMEM((bs,), jnp.int32),
            pltpu.VMEM((bs,), jnp.float32),
            pltpu.VMEM((zl,), jnp.float32),
            pltpu.VMEM_SHARED((t,), jnp.float32),
        ],
    )
    def k(dst_hbm, out_hbm, idx_v, ones_v, zbuf, acc_sh):
        c = lax.axis_index("c")
        s = lax.axis_index("s")
        wid = _wid()
        _zero_vmem(zbuf, zl)
        _zero_vmem(ones_v, bs)
        ones = jnp.ones((16,), jnp.float32)

        def fill(i, _):
            ones_v[pl.ds(i * 16, 16)] = ones
            return 0

        lax.fori_loop(0, bs // 16, fill, 0)
        pltpu.sync_copy(zbuf, acc_sh.at[pl.ds(s * zl, zl)])
        plsc.subcore_barrier()

        def blk(i, _):
            base = wid * ew + i * bs
            pltpu.sync_copy(dst_hbm.at[pl.ds(base, bs)], idx_v)
            pltpu.sync_copy(ones_v, acc_sh.at[idx_v], add=True)
            return 0

        lax.fori_loop(0, nb, blk, 0)
        plsc.subcore_barrier()
        pltpu.sync_copy(acc_sh.at[pl.ds(s * zl, zl)],
                        out_hbm.at[pl.ds(c * t + s * zl, zl)])

    return k


@functools.lru_cache(maxsize=None)
def _make_sc_gcn1(ep, n, t):
    ew, bs, nb = _blocks(ep)
    zl = 4 * t // NSUB

    @functools.partial(
        pl.kernel,
        out_type=jax.ShapeDtypeStruct((2 * 4 * t,), jnp.float32),
        mesh=_get_mesh(),
        compiler_params=_SC_PARAMS,
        scratch_types=[
            pltpu.VMEM((4 * n,), jnp.float32),
            pltpu.VMEM((bs,), jnp.int32),
            pltpu.VMEM((bs,), jnp.int32),
            pltpu.VMEM((bs,), jnp.int32),
            pltpu.VMEM((bs,), jnp.float32),
            pltpu.VMEM((zl,), jnp.float32),
            pltpu.VMEM_SHARED((4 * t,), jnp.float32),
        ],
    )
    def k(x_hbm, src_hbm, dst_hbm, out_hbm,
          xtab, sv, dv, iv, vv, zbuf, acc_sh):
        c = lax.axis_index("c")
        s = lax.axis_index("s")
        wid = _wid()
        pltpu.sync_copy(x_hbm, xtab)
        _zero_vmem(zbuf, zl)
        pltpu.sync_copy(zbuf, acc_sh.at[pl.ds(s * zl, zl)])
        plsc.subcore_barrier()

        def blk(i, _):
            base = wid * ew + i * bs
            pltpu.sync_copy(src_hbm.at[pl.ds(base, bs)], sv)
            pltpu.sync_copy(dst_hbm.at[pl.ds(base, bs)], dv)
            for b in range(4):
                def vec(j, _):
                    svj = sv[pl.ds(j * 16, 16)] + (b * n)
                    vv[pl.ds(j * 16, 16)] = plsc.load_gather(xtab, [svj])
                    iv[pl.ds(j * 16, 16)] = dv[pl.ds(j * 16, 16)] + (b * t)
                    return 0

                lax.fori_loop(0, bs // 16, vec, 0)
                pltpu.sync_copy(vv, acc_sh.at[iv], add=True)
            return 0

        lax.fori_loop(0, nb, blk, 0)
        plsc.subcore_barrier()
        pltpu.sync_copy(acc_sh.at[pl.ds(s * zl, zl)],
                        out_hbm.at[pl.ds(c * 4 * t + s * zl, zl)])

    return k


@functools.lru_cache(maxsize=None)
def _make_sc_segden(ep, t):
    ew, bs, nb = _blocks(ep)
    zl = 4 * t // NSUB
    small = zl < 256
    if small:
        zl = 4 * t

    @functools.partial(
        pl.kernel,
        out_type=jax.ShapeDtypeStruct((2 * 4 * t,), jnp.float32),
        mesh=_get_mesh(),
        compiler_params=_SC_PARAMS,
        scratch_types=[
            pltpu.VMEM((bs,), jnp.int32),
            pltpu.VMEM((bs,), jnp.int32),
            pltpu.VMEM((bs,), jnp.float32),
            pltpu.VMEM((zl,), jnp.float32),
            pltpu.VMEM_SHARED((4 * t,), jnp.float32),
        ],
    )
    def k(vals_hbm, dst_hbm, out_hbm, dv, iv, vv, zbuf, acc_sh):
        c = lax.axis_index("c")
        s = lax.axis_index("s")
        wid = _wid()
        _zero_vmem(zbuf, zl)
        if small:
            @pl.when(s == 0)
            def _():
                pltpu.sync_copy(zbuf, acc_sh)
        else:
            pltpu.sync_copy(zbuf, acc_sh.at[pl.ds(s * zl, zl)])
        plsc.subcore_barrier()

        def blk(i, _):
            base = wid * ew + i * bs
            pltpu.sync_copy(dst_hbm.at[pl.ds(base, bs)], dv)
            for b in range(4):
                pltpu.sync_copy(vals_hbm.at[b, pl.ds(base, bs)], vv)

                def vec(j, _):
                    iv[pl.ds(j * 16, 16)] = dv[pl.ds(j * 16, 16)] + (b * t)
                    return 0

                lax.fori_loop(0, bs // 16, vec, 0)
                pltpu.sync_copy(vv, acc_sh.at[iv], add=True)
            return 0

        lax.fori_loop(0, nb, blk, 0)
        plsc.subcore_barrier()
        if small:
            @pl.when(s == 0)
            def _():
                pltpu.sync_copy(acc_sh, out_hbm.at[pl.ds(c * 4 * t, 4 * t)])
        else:
            pltpu.sync_copy(acc_sh.at[pl.ds(s * zl, zl)],
                            out_hbm.at[pl.ds(c * 4 * t + s * zl, zl)])

    return k


@functools.lru_cache(maxsize=None)
def _make_sc_gatex(ep, t):
    ew, bs, nb = _blocks(ep)

    @functools.partial(
        pl.kernel,
        out_type=jax.ShapeDtypeStruct((4, ep), jnp.float32),
        mesh=_get_mesh(),
        compiler_params=_SC_PARAMS,
        scratch_types=[
            pltpu.VMEM((4 * t,), jnp.float32),
            pltpu.VMEM((4 * t,), jnp.float32),
            pltpu.VMEM((64,), jnp.float32),
            pltpu.VMEM((bs,), jnp.int32),
            pltpu.VMEM((bs,), jnp.int32),
            pltpu.VMEM((bs,), jnp.float32),
        ],
    )
    def k(hs_hbm, hd_hbm, c_hbm, src_hbm, dst_hbm, out_hbm,
          ts, td, tc, sv, dv, ov):
        wid = _wid()
        pltpu.sync_copy(hs_hbm, ts)
        pltpu.sync_copy(hd_hbm, td)
        pltpu.sync_copy(c_hbm, tc)

        def blk(i, _):
            base = wid * ew + i * bs
            pltpu.sync_copy(src_hbm.at[pl.ds(base, bs)], sv)
            pltpu.sync_copy(dst_hbm.at[pl.ds(base, bs)], dv)
            for b in range(4):
                cb = tc[pl.ds(b * 16, 16)]

                def vec(j, _):
                    sl = pl.ds(j * 16, 16)
                    a = plsc.load_gather(ts, [sv[sl] + (b * t)])
                    d = plsc.load_gather(td, [dv[sl] + (b * t)])
                    e = a + d
                    e = jnp.where(e >= 0.0, e, 0.2 * e)
                    ov[sl] = jnp.exp(e - cb)
                    return 0

                lax.fori_loop(0, bs // 16, vec, 0)
                pltpu.sync_copy(ov, out_hbm.at[b, pl.ds(base, bs)])
            return 0

        lax.fori_loop(0, nb, blk, 0)

    return k


@functools.lru_cache(maxsize=None)
def _make_sc_rows(ep, n, mp, scaled):
    ew, bs, nb = _blocks(ep)
    rpt = mp // NSUB

    scratch = [
        pltpu.VMEM((bs,), jnp.int32),
        pltpu.VMEM((bs,), jnp.int32),
        pltpu.VMEM((bs,), jnp.int32),
        pltpu.VMEM((bs,), jnp.float32),
        pltpu.VMEM((bs, 128), jnp.float32),
        pltpu.VMEM((bs, 128), jnp.float32),
        pltpu.VMEM_SHARED((mp, 128), jnp.float32),
        pltpu.SemaphoreType.DMA,
        pltpu.SemaphoreType.DMA,
    ]

    def body(table_hbm, src_hbm, dst_hbm, w_hbm, out_hbm,
             sv0, sv1, dv, wv, r0, r1, acc_sh, sem0, sem1):
        c = lax.axis_index("c")
        s = lax.axis_index("s")
        wid = _wid()
        svs, rs, sems = (sv0, sv1), (r0, r1), (sem0, sem1)

        zb = min(bs, 128)
        _zero_vmem2d(r0, zb)
        nf, zt = rpt // zb, rpt % zb
        for tzi in range(nf):
            pltpu.sync_copy(r0.at[pl.ds(0, zb)],
                            acc_sh.at[pl.ds(s * rpt + tzi * zb, zb)])
        if zt:
            pltpu.sync_copy(r0.at[pl.ds(0, zt)],
                            acc_sh.at[pl.ds(s * rpt + nf * zb, zt)])
        plsc.subcore_barrier()

        pltpu.sync_copy(src_hbm.at[pl.ds(wid * ew, bs)], sv0)
        descs = [pltpu.async_copy(table_hbm.at[sv0], r0, sem0), None]
        for i in range(nb):
            cur = i & 1
            nxt = 1 - cur
            if i + 1 < nb:
                nbase = wid * ew + (i + 1) * bs
                pltpu.sync_copy(src_hbm.at[pl.ds(nbase, bs)], svs[nxt])
                descs[nxt] = pltpu.async_copy(table_hbm.at[svs[nxt]],
                                              rs[nxt], sems[nxt])
            base = wid * ew + i * bs
            pltpu.sync_copy(dst_hbm.at[pl.ds(base, bs)], dv)
            if scaled:
                pltpu.sync_copy(w_hbm.at[pl.ds(base, bs)], wv)
            descs[cur].wait()
            if scaled:
                def scale(r, _, _rs=rs[cur]):
                    wr = plsc.load_gather(wv, [jnp.full((16,), r, jnp.int32)])
                    for j in range(8):
                        sl = pl.ds(j * 16, 16)
                        _rs[r, sl] = _rs[r, sl] * wr
                    return 0
                lax.fori_loop(0, bs, scale, 0)
            pltpu.sync_copy(rs[cur], acc_sh.at[dv], add=True)
        plsc.subcore_barrier()

        base_o = c * mp + s * rpt
        nco = (rpt + 127) // 128
        for t2 in range(nco):
            sz = min(128, rpt - t2 * 128)
            pltpu.sync_copy(acc_sh.at[pl.ds(s * rpt + t2 * 128, sz)],
                            out_hbm.at[pl.ds(base_o + t2 * 128, sz)])

    if scaled:
        def body_s(table_hbm, src_hbm, dst_hbm, w_hbm, out_hbm, *scr):
            body(table_hbm, src_hbm, dst_hbm, w_hbm, out_hbm, *scr)
        fn = body_s
    else:
        def body_n(table_hbm, src_hbm, dst_hbm, out_hbm, *scr):
            body(table_hbm, src_hbm, dst_hbm, None, out_hbm, *scr)
        fn = body_n

    return pl.kernel(
        fn,
        out_type=jax.ShapeDtypeStruct((2 * mp, 128), jnp.float32),
        mesh=_get_mesh(),
        compiler_params=_SC_PARAMS,
        scratch_types=scratch,
    )


def _tc_dinv(degp, xp):
    def body(d_ref, x_ref, o_ref, xs_ref):
        d = d_ref[0] + d_ref[1] + 1.0
        di = lax.rsqrt(d.reshape(80, 128))
        o_ref[...] = di
        dflat = di.reshape(1, NP1)
        xs_ref[...] = (x_ref[...].reshape(BN, NP1)
                       * dflat).reshape(1, BN * NP1)

    dinv, xs = pl.pallas_call(
        body,
        out_shape=[jax.ShapeDtypeStruct((80, 128), jnp.float32),
                   jax.ShapeDtypeStruct((1, BN * NP1), jnp.float32)],
    )(degp.reshape(2, 80, 128), xp.reshape(1, BN * NP1))
    return dinv.reshape(NP1), xs.reshape(BN * NP1)


def _tc_gcn1ln(alphap, x, dinv, w1, g, b):
    blk = 512
    nj = NP1 // blk

    def body(a_ref, x_ref, di_ref, w_ref, g_ref, b_ref, o_ref):
        di = di_ref[0]
        alpha = di * (a_ref[0] + a_ref[1] + di * x_ref[0])
        w = w_ref[0]
        mw = jnp.mean(w)
        vw = jnp.mean((w - mw) ** 2)
        wc = w - mw
        denom = lax.rsqrt(alpha * alpha * vw + EPS)
        o_ref[0] = ((alpha * denom)[:, None] * wc[None, :] * g_ref[0][None, :]
                    + b_ref[0][None, :])

    grid = (BN, nj)
    return pl.pallas_call(
        body,
        grid=grid,
        in_specs=[
            pl.BlockSpec((2, blk), lambda b2, j: (0, b2 * nj + j)),
            pl.BlockSpec((1, blk), lambda b2, j: (0, b2 * nj + j)),
            pl.BlockSpec((1, blk), lambda b2, j: (0, j)),
            pl.BlockSpec((1, HID), lambda b2, j: (0, 0)),
            pl.BlockSpec((1, HID), lambda b2, j: (0, 0)),
            pl.BlockSpec((1, HID), lambda b2, j: (0, 0)),
        ],
        out_specs=pl.BlockSpec((1, blk, HID), lambda b2, j: (b2, j, 0)),
        out_shape=jax.ShapeDtypeStruct((BN, NP1, HID), jnp.float32),
    )(alphap.reshape(2, BN * NP1), x.reshape(1, BN * NP1),
      dinv.reshape(1, NP1), w1.reshape(1, HID),
      g.reshape(1, HID), b.reshape(1, HID))


def _tc_matmul(h, w):
    m, kdim = h.shape
    kout = w.shape[1]
    blk = 512

    def body(h_ref, w_ref, o_ref):
        o_ref[...] = jnp.dot(h_ref[...], w_ref[...],
                             preferred_element_type=jnp.float32)

    return pl.pallas_call(
        body,
        grid=(m // blk,),
        in_specs=[
            pl.BlockSpec((blk, kdim), lambda i: (i, 0)),
            pl.BlockSpec((kdim, kout), lambda i: (0, 0)),
        ],
        out_specs=pl.BlockSpec((blk, kout), lambda i: (i, 0)),
        out_shape=jax.ShapeDtypeStruct((m, kout), jnp.float32),
    )(h, w)


def _tc_matmul_scale(h, w, sc):
    m, kdim = h.shape
    kout = w.shape[1]
    blk = 512

    def body(h_ref, w_ref, s_ref, o_ref):
        o_ref[...] = jnp.dot(h_ref[...], w_ref[...],
                             preferred_element_type=jnp.float32) \
            * s_ref[0][:, None]

    return pl.pallas_call(
        body,
        grid=(m // blk,),
        in_specs=[
            pl.BlockSpec((blk, kdim), lambda i: (i, 0)),
            pl.BlockSpec((kdim, kout), lambda i: (0, 0)),
            pl.BlockSpec((1, blk), lambda i: (0, i)),
        ],
        out_specs=pl.BlockSpec((blk, kout), lambda i: (i, 0)),
        out_shape=jax.ShapeDtypeStruct((m, kout), jnp.float32),
    )(h, w, sc.reshape(1, m))


def _tc_scalerows(h, ex):
    bdim, ndim = h.shape[0], h.shape[1]
    blk = 512
    nj = ndim // blk

    def body(h_ref, e_ref, o_ref):
        o_ref[0] = h_ref[0] * e_ref[0][:, None]

    return pl.pallas_call(
        body,
        grid=(bdim, nj),
        in_specs=[
            pl.BlockSpec((1, blk, HID), lambda b, j: (b, j, 0)),
            pl.BlockSpec((1, blk), lambda b, j: (0, b * nj + j)),
        ],
        out_specs=pl.BlockSpec((1, blk, HID), lambda b, j: (b, j, 0)),
        out_shape=jax.ShapeDtypeStruct((bdim, ndim, HID), jnp.float32),
    )(h, ex.reshape(1, bdim * ndim))


def _tc_gcn2post(aggp, h2, h1, dinv, b2, g, b):
    blk = 512

    def body(a_ref, h2_ref, h1_ref, di_ref, b2_ref, g_ref, b_ref, o_ref):
        pre = (di_ref[0][:, None] * (a_ref[0, 0] + a_ref[1, 0] + h2_ref[0])
               + b2_ref[0][None, :])
        m = jnp.mean(pre, axis=-1, keepdims=True)
        v = jnp.mean((pre - m) ** 2, axis=-1, keepdims=True)
        o_ref[0] = h1_ref[0] + ((pre - m) * lax.rsqrt(v + EPS)
                                * g_ref[0][None, :] + b_ref[0][None, :])

    grid = (BN, NP1 // blk)
    return pl.pallas_call(
        body,
        grid=grid,
        in_specs=[
            pl.BlockSpec((2, 1, blk, HID), lambda b3, j: (0, b3, j, 0)),
            pl.BlockSpec((1, blk, HID), lambda b3, j: (b3, j, 0)),
            pl.BlockSpec((1, blk, HID), lambda b3, j: (b3, j, 0)),
            pl.BlockSpec((1, blk), lambda b3, j: (0, j)),
            pl.BlockSpec((1, HID), lambda b3, j: (0, 0)),
            pl.BlockSpec((1, HID), lambda b3, j: (0, 0)),
            pl.BlockSpec((1, HID), lambda b3, j: (0, 0)),
        ],
        out_specs=pl.BlockSpec((1, blk, HID), lambda b3, j: (b3, j, 0)),
        out_shape=jax.ShapeDtypeStruct((BN, NP1, HID), jnp.float32),
    )(aggp, h2, h1, dinv.reshape(1, NP1), b2.reshape(1, HID),
      g.reshape(1, HID), b.reshape(1, HID))


def _tc_s2t_scores(hflat, p):
    m = hflat.shape[0]
    blk = 512
    f1w, f1b = p['fc1_W'], p['fc1_b']
    lng, lnb = p['ln_g'], p['ln_b']
    f2w = p['fc2_W'][:, 0]
    f2b = p['fc2_b'][0]

    def body(h_ref, w1_ref, b1_ref, g_ref, b_ref, w2_ref, b2_ref, o_ref):
        z = jnp.dot(h_ref[...], w1_ref[...],
                    preferred_element_type=jnp.float32) + b1_ref[0][None, :]
        mu = jnp.mean(z, axis=-1, keepdims=True)
        v = jnp.mean((z - mu) ** 2, axis=-1, keepdims=True)
        t = jnp.tanh((z - mu) * lax.rsqrt(v + EPS) * g_ref[0][None, :]
                     + b_ref[0][None, :])
        o_ref[0] = jnp.sum(t * w2_ref[0][None, :], axis=-1) + b2_ref[0, 0]

    return pl.pallas_call(
        body,
        grid=(m // blk,),
        in_specs=[
            pl.BlockSpec((blk, HID), lambda i: (i, 0)),
            pl.BlockSpec((HID, 4 * HID), lambda i: (0, 0)),
            pl.BlockSpec((1, 4 * HID), lambda i: (0, 0)),
            pl.BlockSpec((1, 4 * HID), lambda i: (0, 0)),
            pl.BlockSpec((1, 4 * HID), lambda i: (0, 0)),
            pl.BlockSpec((1, 4 * HID), lambda i: (0, 0)),
            pl.BlockSpec((1, 1), lambda i: (0, 0)),
        ],
        out_specs=pl.BlockSpec((1, blk), lambda i: (0, i)),
        out_shape=jax.ShapeDtypeStruct((1, m), jnp.float32),
    )(hflat, f1w, f1b.reshape(1, -1), lng.reshape(1, -1),
      lnb.reshape(1, -1), f2w.reshape(1, -1),
      p['fc2_b'].reshape(1, 1)).reshape(m)


def _tc_sexp(s, npad, nvalid):
    def body(s_ref, o_ref):
        row = s_ref[0]
        iot = lax.broadcasted_iota(jnp.int32, (npad,), 0)
        msk = iot < nvalid
        sm = jnp.where(msk, row, -1e30)
        cmax = jnp.max(sm)
        o_ref[0] = jnp.where(msk, jnp.exp(row - cmax), 0.0)

    return pl.pallas_call(
        body,
        grid=(BN,),
        in_specs=[pl.BlockSpec((1, npad), lambda b: (0, b))],
        out_specs=pl.BlockSpec((1, npad), lambda b: (0, b)),
        out_shape=jax.ShapeDtypeStruct((1, BN * npad), jnp.float32),
    )(s.reshape(1, BN * npad)).reshape(BN, npad)


def _tc_gatpre(lv2p, d0, d1, gw, a_s, a_d, nvalid):
    def body(l_ref, d0_ref, d1_ref, w_ref, as_ref, ad_ref, hw_ref, hs_ref,
             hd_ref, c_ref, es_ref):
        den = d0_ref[0] + d1_ref[0] + 1e-16
        lv2 = (l_ref[0, 0] + l_ref[1, 0]) / den[:, None]
        hw = jnp.dot(lv2, w_ref[...], preferred_element_type=jnp.float32)
        hs = jnp.sum(hw * as_ref[0][None, :], axis=-1)
        hd = jnp.sum(hw * ad_ref[0][None, :], axis=-1)
        iot = lax.broadcasted_iota(jnp.int32, (NP2,), 0)
        msk = iot < nvalid
        hs = jnp.where(msk, hs, 0.0)
        hd = jnp.where(msk, hd, 0.0)
        cb = jnp.maximum(jnp.max(jnp.where(msk, hs, -1e30))
                         + jnp.max(jnp.where(msk, hd, -1e30)), 0.0)
        e = hs + hd
        e = jnp.where(e >= 0.0, e, 0.2 * e)
        hw_ref[0] = hw
        hs_ref[0] = hs
        hd_ref[0] = hd
        c_ref[0] = jnp.full((NP2,), cb, jnp.float32)
        es_ref[0] = jnp.where(msk, jnp.exp(e - cb), 0.0)

    outs = pl.pallas_call(
        body,
        grid=(BN,),
        in_specs=[
            pl.BlockSpec((2, 1, NP2, HID), lambda b: (0, b, 0, 0)),
            pl.BlockSpec((1, NP2), lambda b: (0, b)),
            pl.BlockSpec((1, NP2), lambda b: (0, b)),
            pl.BlockSpec((HID, HID), lambda b: (0, 0)),
            pl.BlockSpec((1, HID), lambda b: (0, 0)),
            pl.BlockSpec((1, HID), lambda b: (0, 0)),
        ],
        out_specs=[
            pl.BlockSpec((1, NP2, HID), lambda b: (b, 0, 0)),
            pl.BlockSpec((1, NP2), lambda b: (0, b)),
            pl.BlockSpec((1, NP2), lambda b: (0, b)),
            pl.BlockSpec((1, NP2), lambda b: (0, b)),
            pl.BlockSpec((1, NP2), lambda b: (0, b)),
        ],
        out_shape=[
            jax.ShapeDtypeStruct((BN, NP2, HID), jnp.float32),
            jax.ShapeDtypeStruct((1, BN * NP2), jnp.float32),
            jax.ShapeDtypeStruct((1, BN * NP2), jnp.float32),
            jax.ShapeDtypeStruct((1, BN * NP2), jnp.float32),
            jax.ShapeDtypeStruct((1, BN * NP2), jnp.float32),
        ],
    )(lv2p, d0.reshape(1, BN * NP2), d1.reshape(1, BN * NP2), gw,
      a_s.reshape(1, HID), a_d.reshape(1, HID))
    return outs


def _tc_gatpost(gaggp, hw, ex_self, d0, d1, gb, g, b):
    def body(a_ref, hw_ref, es_ref, d0_ref, d1_ref, gb_ref, g_ref, b_ref,
             o_ref):
        es = es_ref[0]
        den = d0_ref[0] + d1_ref[0] + es + 1e-16
        pre = ((a_ref[0, 0] + a_ref[1, 0] + es[:, None] * hw_ref[0])
               / den[:, None] + gb_ref[0][None, :])
        m = jnp.mean(pre, axis=-1, keepdims=True)
        v = jnp.mean((pre - m) ** 2, axis=-1, keepdims=True)
        o_ref[0] = ((pre - m) * lax.rsqrt(v + EPS) * g_ref[0][None, :]
                    + b_ref[0][None, :])

    return pl.pallas_call(
        body,
        grid=(BN,),
        in_specs=[
            pl.BlockSpec((2, 1, NP2, HID), lambda b2: (0, b2, 0, 0)),
            pl.BlockSpec((1, NP2, HID), lambda b2: (b2, 0, 0)),
            pl.BlockSpec((1, NP2), lambda b2: (0, b2)),
            pl.BlockSpec((1, NP2), lambda b2: (0, b2)),
            pl.BlockSpec((1, NP2), lambda b2: (0, b2)),
            pl.BlockSpec((1, HID), lambda b2: (0, 0)),
            pl.BlockSpec((1, HID), lambda b2: (0, 0)),
            pl.BlockSpec((1, HID), lambda b2: (0, 0)),
        ],
        out_specs=pl.BlockSpec((1, NP2, HID), lambda b2: (b2, 0, 0)),
        out_shape=jax.ShapeDtypeStruct((BN, NP2, HID), jnp.float32),
    )(gaggp, hw, ex_self.reshape(1, BN * NP2), d0.reshape(1, BN * NP2),
      d1.reshape(1, BN * NP2), gb.reshape(1, HID),
      g.reshape(1, HID), b.reshape(1, HID))


def _tc_final(parts, d0, d1):
    def body(p_ref, d0_ref, d1_ref, o_ref):
        den = d0_ref[0] + d1_ref[0] + 1e-16
        o_ref[0] = (p_ref[0, 0] + p_ref[1, 0]) / den[:, None]

    return pl.pallas_call(
        body,
        grid=(BN,),
        in_specs=[
            pl.BlockSpec((2, 1, NP3, HID), lambda b: (0, b, 0, 0)),
            pl.BlockSpec((1, NP3), lambda b: (0, b)),
            pl.BlockSpec((1, NP3), lambda b: (0, b)),
        ],
        out_specs=pl.BlockSpec((1, NP3, HID), lambda b: (b, 0, 0)),
        out_shape=jax.ShapeDtypeStruct((BN, NP3, HID), jnp.float32),
    )(parts, d0.reshape(1, BN * NP3), d1.reshape(1, BN * NP3))


def kernel(x, batch_num, level_11_edge_index, level_22_edge_index,
           level_21_seg, level_32_seg, level_13_indicator, params):
    p = params
    B = x.shape[0]

    i32 = jnp.int32
    src1 = level_11_edge_index[0].astype(i32)
    dst1 = level_11_edge_index[1].astype(i32)
    pad1 = jnp.full((EP1 - L1E,), NP1 - 1, i32)
    src1p = jnp.concatenate([src1, pad1])
    dst1p = jnp.concatenate([dst1, pad1])

    src2 = level_22_edge_index[0].astype(i32)
    dst2 = level_22_edge_index[1].astype(i32)
    pad2 = jnp.full((EP2 - L2E,), NP2 - 1, i32)
    src2p = jnp.concatenate([src2, pad2])
    dst2p = jnp.concatenate([dst2, pad2])

    seg21 = level_21_seg.astype(i32)
    seg21p = jnp.concatenate([seg21, jnp.full((SP1 - L1N,), NP2 - 1, i32)])
    iota1p = jnp.minimum(jnp.arange(SP1, dtype=i32), NP1 - 1)

    seg32 = level_32_seg.astype(i32)
    seg32p = jnp.concatenate([seg32, jnp.full((NP2 - L2N,), NP3 - 1, i32)])
    iota2p = jnp.arange(NP2, dtype=i32)

    xp = jnp.pad(x, ((0, 0), (0, NP1 - L1N)))

    degp = _make_sc_deg(EP1, NP1)(dst1p)
    dinv, xs = _tc_dinv(degp, xp)

    alphap = _make_sc_gcn1(EP1, NP1, NP1)(xs, src1p, dst1p)
    h1 = _tc_gcn1ln(alphap.reshape(2, BN, NP1), xp, dinv,
                    p['gcn1_W'][0], p['ln1_g'], p['ln1_b'])

    h1f = h1.reshape(BN * NP1, HID)
    dinv4 = jnp.tile(dinv, BN)
    h2s = _tc_matmul_scale(h1f, p['gcn2_W'], dinv4).reshape(BN, NP1, HID)
    rows1 = _make_sc_rows(EP1, NP1, NP1, False)
    aggp = jnp.stack([rows1(h2s[b], src1p, dst1p).reshape(2, NP1, HID)
                      for b in range(BN)], axis=1)
    h = _tc_gcn2post(aggp, h2s, h1, dinv, p['gcn2_b'],
                     p['ln2_g'], p['ln2_b'])

    s12 = _tc_s2t_scores(h.reshape(BN * NP1, HID),
                         p['s2t12']).reshape(BN, NP1)
    ex12 = _tc_sexp(s12, NP1, L1N)
    ex12p = jnp.pad(ex12, ((0, 0), (0, SP1 - NP1)))
    den12 = _make_sc_segden(SP1, NP2)(ex12p, seg21p)
    d12 = den12.reshape(2, 4 * NP2)
    hs12 = _tc_scalerows(h, ex12)
    rows12 = _make_sc_rows(SP1, NP1, NP2, False)
    lv2p = jnp.stack([rows12(hs12[b], iota1p, seg21p).reshape(2, NP2, HID)
                      for b in range(BN)], axis=1)

    hw, hs, hd, cb, ex_self = _tc_gatpre(lv2p, d12[0], d12[1], p['gat_W'],
                                         p['gat_as'], p['gat_ad'], L2N)
    cb64 = cb.reshape(BN, NP2)[:, :16].reshape(64)
    exg = _make_sc_gatex(EP2, NP2)(hs.reshape(4 * NP2), hd.reshape(4 * NP2),
                                   cb64, src2p, dst2p)
    deng = _make_sc_segden(EP2, NP2)(exg, dst2p)
    dg = deng.reshape(2, 4 * NP2)
    rowsg = _make_sc_rows(EP2, NP2, NP2, True)
    gaggp = jnp.stack([rowsg(hw[b], src2p, dst2p, exg[b]).reshape(2, NP2, HID)
                       for b in range(BN)], axis=1)
    dgb = dg.reshape(2, BN, NP2)
    lv2f = _tc_gatpost(gaggp, hw, ex_self, dgb[0], dgb[1],
                       p['gat_b'], p['lng_g'], p['lng_b'])

    s23 = _tc_s2t_scores(lv2f.reshape(BN * NP2, HID),
                         p['s2t23']).reshape(BN, NP2)
    ex23 = _tc_sexp(s23, NP2, L2N)
    den23 = _make_sc_segden(NP2, NP3)(ex23, seg32p)
    d23 = den23.reshape(2, 4 * NP3)
    hs23 = _tc_scalerows(lv2f, ex23)
    rows23 = _make_sc_rows(NP2, NP2, NP3, False)
    lv3p = jnp.stack([rows23(hs23[b], iota2p, seg32p).reshape(2, NP3, HID)
                      for b in range(BN)], axis=1)
    lv3 = _tc_final(lv3p, d23[0], d23[1])
    return lv3[:, :L3N, :]


# speedup vs baseline: 10.6122x; 1.0891x over previous
import functools
import math

import jax
import jax.numpy as jnp
from jax import lax
from jax.experimental import pallas as pl
from jax.experimental.pallas import tpu as pltpu
from jax.experimental.pallas import tpu_sc as plsc

L1N = 10000; L2N = 1000; L3N = 50; L1E = 160000; L2E = 16000; HID = 128; BN = 4

NP1 = 10240
NP2 = 1024
NP3 = 1024
EP1 = 163840
EP2 = 16384
SP1 = 12288
NW = 32
NSUB = 16
EPS = 1e-5
_STOP_AT = 99

_mesh = None
_SC_PARAMS = pltpu.CompilerParams(needs_layout_passes=False)


def _get_mesh():
    global _mesh
    if _mesh is None:
        _mesh = plsc.VectorSubcoreMesh(core_axis_name="c", subcore_axis_name="s",
                                       num_cores=2, num_subcores=NSUB)
    return _mesh


def _wid():
    return lax.axis_index("s") * 2 + lax.axis_index("c")


def _zero_vmem(ref, n):
    z = jnp.zeros((16,), jnp.float32)

    def body(i, _):
        ref[pl.ds(i * 16, 16)] = z
        return 0

    lax.fori_loop(0, n // 16, body, 0)


def _zero_vmem2d(ref, rows):
    z = jnp.zeros((16,), jnp.float32)

    def body(i, _):
        for j in range(8):
            ref[i, pl.ds(j * 16, 16)] = z
        return 0

    lax.fori_loop(0, rows, body, 0)


def _blocks(ep):
    ew = ep // NW
    if ew >= 128:
        assert ew % 128 == 0
        return ew, 128, ew // 128
    assert ew % 16 == 0
    return ew, ew, 1


@functools.lru_cache(maxsize=None)
def _make_sc_deg(ep, t):
    ew, bs, nb = _blocks(ep)
    zl = t // NSUB

    @functools.partial(
        pl.kernel,
        out_type=jax.ShapeDtypeStruct((2 * t,), jnp.float32),
        mesh=_get_mesh(),
        compiler_params=_SC_PARAMS,
        scratch_types=[
            pltpu.VMEM((bs,), jnp.int32),
            pltpu.VMEM((bs,), jnp.float32),
            pltpu.VMEM((zl,), jnp.float32),
            pltpu.VMEM_SHARED((t,), jnp.float32),
        ],
    )
    def k(dst_hbm, out_hbm, idx_v, ones_v, zbuf, acc_sh):
        c = lax.axis_index("c")
        s = lax.axis_index("s")
        wid = _wid()
        _zero_vmem(zbuf, zl)
        _zero_vmem(ones_v, bs)
        ones = jnp.ones((16,), jnp.float32)

        def fill(i, _):
            ones_v[pl.ds(i * 16, 16)] = ones
            return 0

        lax.fori_loop(0, bs // 16, fill, 0)
        pltpu.sync_copy(zbuf, acc_sh.at[pl.ds(s * zl, zl)])
        plsc.subcore_barrier()

        def blk(i, _):
            base = wid * ew + i * bs
            pltpu.sync_copy(dst_hbm.at[pl.ds(base, bs)], idx_v)
            pltpu.sync_copy(ones_v, acc_sh.at[idx_v], add=True)
            return 0

        lax.fori_loop(0, nb, blk, 0)
        plsc.subcore_barrier()
        pltpu.sync_copy(acc_sh.at[pl.ds(s * zl, zl)],
                        out_hbm.at[pl.ds(c * t + s * zl, zl)])

    return k


@functools.lru_cache(maxsize=None)
def _make_sc_gcn1(ep, n, t):
    ew, bs, nb = _blocks(ep)
    zl = 4 * t // NSUB

    @functools.partial(
        pl.kernel,
        out_type=jax.ShapeDtypeStruct((2 * 4 * t,), jnp.float32),
        mesh=_get_mesh(),
        compiler_params=_SC_PARAMS,
        scratch_types=[
            pltpu.VMEM((4 * n,), jnp.float32),
            pltpu.VMEM((bs,), jnp.int32),
            pltpu.VMEM((bs,), jnp.int32),
            pltpu.VMEM((bs,), jnp.int32),
            pltpu.VMEM((bs,), jnp.float32),
            pltpu.VMEM((zl,), jnp.float32),
            pltpu.VMEM_SHARED((4 * t,), jnp.float32),
        ],
    )
    def k(x_hbm, src_hbm, dst_hbm, out_hbm,
          xtab, sv, dv, iv, vv, zbuf, acc_sh):
        c = lax.axis_index("c")
        s = lax.axis_index("s")
        wid = _wid()
        pltpu.sync_copy(x_hbm, xtab)
        _zero_vmem(zbuf, zl)
        pltpu.sync_copy(zbuf, acc_sh.at[pl.ds(s * zl, zl)])
        plsc.subcore_barrier()

        def blk(i, _):
            base = wid * ew + i * bs
            pltpu.sync_copy(src_hbm.at[pl.ds(base, bs)], sv)
            pltpu.sync_copy(dst_hbm.at[pl.ds(base, bs)], dv)
            for b in range(4):
                def vec(j, _):
                    svj = sv[pl.ds(j * 16, 16)] + (b * n)
                    vv[pl.ds(j * 16, 16)] = plsc.load_gather(xtab, [svj])
                    iv[pl.ds(j * 16, 16)] = dv[pl.ds(j * 16, 16)] + (b * t)
                    return 0

                lax.fori_loop(0, bs // 16, vec, 0)
                pltpu.sync_copy(vv, acc_sh.at[iv], add=True)
            return 0

        lax.fori_loop(0, nb, blk, 0)
        plsc.subcore_barrier()
        pltpu.sync_copy(acc_sh.at[pl.ds(s * zl, zl)],
                        out_hbm.at[pl.ds(c * 4 * t + s * zl, zl)])

    return k


@functools.lru_cache(maxsize=None)
def _make_sc_segden(ep, t):
    ew, bs, nb = _blocks(ep)
    zl = 4 * t // NSUB
    small = zl < 256
    if small:
        zl = 4 * t

    @functools.partial(
        pl.kernel,
        out_type=jax.ShapeDtypeStruct((2 * 4 * t,), jnp.float32),
        mesh=_get_mesh(),
        compiler_params=_SC_PARAMS,
        scratch_types=[
            pltpu.VMEM((bs,), jnp.int32),
            pltpu.VMEM((bs,), jnp.int32),
            pltpu.VMEM((bs,), jnp.float32),
            pltpu.VMEM((zl,), jnp.float32),
            pltpu.VMEM_SHARED((4 * t,), jnp.float32),
        ],
    )
    def k(vals_hbm, dst_hbm, out_hbm, dv, iv, vv, zbuf, acc_sh):
        c = lax.axis_index("c")
        s = lax.axis_index("s")
        wid = _wid()
        _zero_vmem(zbuf, zl)
        if small:
            @pl.when(s == 0)
            def _():
                pltpu.sync_copy(zbuf, acc_sh)
        else:
            pltpu.sync_copy(zbuf, acc_sh.at[pl.ds(s * zl, zl)])
        plsc.subcore_barrier()

        def blk(i, _):
            base = wid * ew + i * bs
            pltpu.sync_copy(dst_hbm.at[pl.ds(base, bs)], dv)
            for b in range(4):
                pltpu.sync_copy(vals_hbm.at[b, pl.ds(base, bs)], vv)

                def vec(j, _):
                    iv[pl.ds(j * 16, 16)] = dv[pl.ds(j * 16, 16)] + (b * t)
                    return 0

                lax.fori_loop(0, bs // 16, vec, 0)
                pltpu.sync_copy(vv, acc_sh.at[iv], add=True)
            return 0

        lax.fori_loop(0, nb, blk, 0)
        plsc.subcore_barrier()
        if small:
            @pl.when(s == 0)
            def _():
                pltpu.sync_copy(acc_sh, out_hbm.at[pl.ds(c * 4 * t, 4 * t)])
        else:
            pltpu.sync_copy(acc_sh.at[pl.ds(s * zl, zl)],
                            out_hbm.at[pl.ds(c * 4 * t + s * zl, zl)])

    return k


@functools.lru_cache(maxsize=None)
def _make_sc_gatex(ep, t):
    ew, bs, nb = _blocks(ep)

    @functools.partial(
        pl.kernel,
        out_type=jax.ShapeDtypeStruct((4, ep), jnp.float32),
        mesh=_get_mesh(),
        compiler_params=_SC_PARAMS,
        scratch_types=[
            pltpu.VMEM((4 * t,), jnp.float32),
            pltpu.VMEM((4 * t,), jnp.float32),
            pltpu.VMEM((64,), jnp.float32),
            pltpu.VMEM((bs,), jnp.int32),
            pltpu.VMEM((bs,), jnp.int32),
            pltpu.VMEM((bs,), jnp.float32),
        ],
    )
    def k(hs_hbm, hd_hbm, c_hbm, src_hbm, dst_hbm, out_hbm,
          ts, td, tc, sv, dv, ov):
        wid = _wid()
        pltpu.sync_copy(hs_hbm, ts)
        pltpu.sync_copy(hd_hbm, td)
        pltpu.sync_copy(c_hbm, tc)

        def blk(i, _):
            base = wid * ew + i * bs
            pltpu.sync_copy(src_hbm.at[pl.ds(base, bs)], sv)
            pltpu.sync_copy(dst_hbm.at[pl.ds(base, bs)], dv)
            for b in range(4):
                cb = tc[pl.ds(b * 16, 16)]

                def vec(j, _):
                    sl = pl.ds(j * 16, 16)
                    a = plsc.load_gather(ts, [sv[sl] + (b * t)])
                    d = plsc.load_gather(td, [dv[sl] + (b * t)])
                    e = a + d
                    e = jnp.where(e >= 0.0, e, 0.2 * e)
                    ov[sl] = jnp.exp(e - cb)
                    return 0

                lax.fori_loop(0, bs // 16, vec, 0)
                pltpu.sync_copy(ov, out_hbm.at[b, pl.ds(base, bs)])
            return 0

        lax.fori_loop(0, nb, blk, 0)

    return k


@functools.lru_cache(maxsize=None)
def _make_sc_rows(ep, n, mp, scaled):
    ew, bs, nb = _blocks(ep)
    rpt = mp // NSUB

    scratch = [
        pltpu.VMEM((bs,), jnp.int32),
        pltpu.VMEM((bs,), jnp.int32),
        pltpu.VMEM((bs,), jnp.int32),
        pltpu.VMEM((bs,), jnp.float32),
        pltpu.VMEM((bs, 128), jnp.float32),
        pltpu.VMEM((bs, 128), jnp.float32),
        pltpu.VMEM_SHARED((mp, 128), jnp.float32),
        pltpu.SemaphoreType.DMA,
        pltpu.SemaphoreType.DMA,
    ]

    def body(tabs, src_hbm, dst_hbm, w_hbm, out_hbm,
             sv0, sv1, dv, wv, r0, r1, acc_sh, sem0, sem1):
        c = lax.axis_index("c")
        s = lax.axis_index("s")
        wid = _wid()
        svs, rs, sems = (sv0, sv1), (r0, r1), (sem0, sem1)
        zb = min(bs, 128)
        nf, zt = rpt // zb, rpt % zb
        nco = (rpt + 127) // 128

        for bnum in range(4):
            tab = tabs[bnum]
            _zero_vmem2d(r0, zb)
            for tzi in range(nf):
                pltpu.sync_copy(r0.at[pl.ds(0, zb)],
                                acc_sh.at[pl.ds(s * rpt + tzi * zb, zb)])
            if zt:
                pltpu.sync_copy(r0.at[pl.ds(0, zt)],
                                acc_sh.at[pl.ds(s * rpt + nf * zb, zt)])
            plsc.subcore_barrier()

            pltpu.sync_copy(src_hbm.at[pl.ds(wid * ew, bs)], sv0)
            descs = [pltpu.async_copy(tab.at[sv0], r0, sem0), None]
            for i in range(nb):
                cur = i & 1
                nxt = 1 - cur
                if i + 1 < nb:
                    nbase = wid * ew + (i + 1) * bs
                    pltpu.sync_copy(src_hbm.at[pl.ds(nbase, bs)], svs[nxt])
                    descs[nxt] = pltpu.async_copy(tab.at[svs[nxt]],
                                                  rs[nxt], sems[nxt])
                base = wid * ew + i * bs
                pltpu.sync_copy(dst_hbm.at[pl.ds(base, bs)], dv)
                if scaled:
                    pltpu.sync_copy(w_hbm.at[bnum, pl.ds(base, bs)], wv)
                descs[cur].wait()
                if scaled:
                    def scale(r, _, _rs=rs[cur]):
                        wr = plsc.load_gather(
                            wv, [jnp.full((16,), r, jnp.int32)])
                        for j in range(8):
                            sl = pl.ds(j * 16, 16)
                            _rs[r, sl] = _rs[r, sl] * wr
                        return 0
                    lax.fori_loop(0, bs, scale, 0)
                pltpu.sync_copy(rs[cur], acc_sh.at[dv], add=True)
            plsc.subcore_barrier()

            base_o = (2 * bnum + c) * mp + s * rpt
            for t2 in range(nco):
                sz = min(128, rpt - t2 * 128)
                pltpu.sync_copy(acc_sh.at[pl.ds(s * rpt + t2 * 128, sz)],
                                out_hbm.at[pl.ds(base_o + t2 * 128, sz)])

    if scaled:
        def body_s(t0, t1, t2, t3, src_hbm, dst_hbm, w_hbm, out_hbm, *scr):
            body((t0, t1, t2, t3), src_hbm, dst_hbm, w_hbm, out_hbm, *scr)
        fn = body_s
    else:
        def body_n(t0, t1, t2, t3, src_hbm, dst_hbm, out_hbm, *scr):
            body((t0, t1, t2, t3), src_hbm, dst_hbm, None, out_hbm, *scr)
        fn = body_n

    return pl.kernel(
        fn,
        out_type=jax.ShapeDtypeStruct((4 * 2 * mp, 128), jnp.float32),
        mesh=_get_mesh(),
        compiler_params=_SC_PARAMS,
        scratch_types=scratch,
    )


def _tc_dinv(degp, xp):
    def body(d_ref, x_ref, o_ref, xs_ref):
        d = d_ref[0] + d_ref[1] + 1.0
        di = lax.rsqrt(d.reshape(80, 128))
        o_ref[...] = di
        dflat = di.reshape(1, NP1)
        xs_ref[...] = (x_ref[...].reshape(BN, NP1)
                       * dflat).reshape(1, BN * NP1)

    dinv, xs = pl.pallas_call(
        body,
        out_shape=[jax.ShapeDtypeStruct((80, 128), jnp.float32),
                   jax.ShapeDtypeStruct((1, BN * NP1), jnp.float32)],
    )(degp.reshape(2, 80, 128), xp.reshape(1, BN * NP1))
    return dinv.reshape(NP1), xs.reshape(BN * NP1)


def _tc_gcn1ln(alphap, x, dinv, w1, g, b):
    blk = 512
    nj = NP1 // blk

    def body(a_ref, x_ref, di_ref, w_ref, g_ref, b_ref, o_ref):
        di = di_ref[0]
        alpha = di * (a_ref[0] + a_ref[1] + di * x_ref[0])
        w = w_ref[0]
        mw = jnp.mean(w)
        vw = jnp.mean((w - mw) ** 2)
        wc = w - mw
        denom = lax.rsqrt(alpha * alpha * vw + EPS)
        o_ref[0] = ((alpha * denom)[:, None] * wc[None, :] * g_ref[0][None, :]
                    + b_ref[0][None, :])

    grid = (BN, nj)
    return pl.pallas_call(
        body,
        grid=grid,
        in_specs=[
            pl.BlockSpec((2, blk), lambda b2, j: (0, b2 * nj + j)),
            pl.BlockSpec((1, blk), lambda b2, j: (0, b2 * nj + j)),
            pl.BlockSpec((1, blk), lambda b2, j: (0, j)),
            pl.BlockSpec((1, HID), lambda b2, j: (0, 0)),
            pl.BlockSpec((1, HID), lambda b2, j: (0, 0)),
            pl.BlockSpec((1, HID), lambda b2, j: (0, 0)),
        ],
        out_specs=pl.BlockSpec((1, blk, HID), lambda b2, j: (b2, j, 0)),
        out_shape=jax.ShapeDtypeStruct((BN, NP1, HID), jnp.float32),
    )(alphap.reshape(2, BN * NP1), x.reshape(1, BN * NP1),
      dinv.reshape(1, NP1), w1.reshape(1, HID),
      g.reshape(1, HID), b.reshape(1, HID))


def _tc_matmul(h, w):
    m, kdim = h.shape
    kout = w.shape[1]
    blk = 512

    def body(h_ref, w_ref, o_ref):
        o_ref[...] = jnp.dot(h_ref[...], w_ref[...],
                             preferred_element_type=jnp.float32)

    return pl.pallas_call(
        body,
        grid=(m // blk,),
        in_specs=[
            pl.BlockSpec((blk, kdim), lambda i: (i, 0)),
            pl.BlockSpec((kdim, kout), lambda i: (0, 0)),
        ],
        out_specs=pl.BlockSpec((blk, kout), lambda i: (i, 0)),
        out_shape=jax.ShapeDtypeStruct((m, kout), jnp.float32),
    )(h, w)


def _tc_matmul_scale(h, w, sc):
    m, kdim = h.shape
    kout = w.shape[1]
    blk = 512

    def body(h_ref, w_ref, s_ref, o_ref):
        o_ref[...] = jnp.dot(h_ref[...], w_ref[...],
                             preferred_element_type=jnp.float32) \
            * s_ref[0][:, None]

    return pl.pallas_call(
        body,
        grid=(m // blk,),
        in_specs=[
            pl.BlockSpec((blk, kdim), lambda i: (i, 0)),
            pl.BlockSpec((kdim, kout), lambda i: (0, 0)),
            pl.BlockSpec((1, blk), lambda i: (0, i)),
        ],
        out_specs=pl.BlockSpec((blk, kout), lambda i: (i, 0)),
        out_shape=jax.ShapeDtypeStruct((m, kout), jnp.float32),
    )(h, w, sc.reshape(1, m))


def _tc_scalerows(h, ex):
    bdim, ndim = h.shape[0], h.shape[1]
    blk = 512
    nj = ndim // blk

    def body(h_ref, e_ref, o_ref):
        o_ref[0] = h_ref[0] * e_ref[0][:, None]

    return pl.pallas_call(
        body,
        grid=(bdim, nj),
        in_specs=[
            pl.BlockSpec((1, blk, HID), lambda b, j: (b, j, 0)),
            pl.BlockSpec((1, blk), lambda b, j: (0, b * nj + j)),
        ],
        out_specs=pl.BlockSpec((1, blk, HID), lambda b, j: (b, j, 0)),
        out_shape=jax.ShapeDtypeStruct((bdim, ndim, HID), jnp.float32),
    )(h, ex.reshape(1, bdim * ndim))


def _tc_gcn2post(aggp, h2, h1, dinv, b2, g, b):
    blk = 512

    def body(a_ref, h2_ref, h1_ref, di_ref, b2_ref, g_ref, b_ref, o_ref):
        pre = (di_ref[0][:, None] * (a_ref[0, 0] + a_ref[0, 1] + h2_ref[0])
               + b2_ref[0][None, :])
        m = jnp.mean(pre, axis=-1, keepdims=True)
        v = jnp.mean((pre - m) ** 2, axis=-1, keepdims=True)
        o_ref[0] = h1_ref[0] + ((pre - m) * lax.rsqrt(v + EPS)
                                * g_ref[0][None, :] + b_ref[0][None, :])

    grid = (BN, NP1 // blk)
    return pl.pallas_call(
        body,
        grid=grid,
        in_specs=[
            pl.BlockSpec((1, 2, blk, HID), lambda b3, j: (b3, 0, j, 0)),
            pl.BlockSpec((1, blk, HID), lambda b3, j: (b3, j, 0)),
            pl.BlockSpec((1, blk, HID), lambda b3, j: (b3, j, 0)),
            pl.BlockSpec((1, blk), lambda b3, j: (0, j)),
            pl.BlockSpec((1, HID), lambda b3, j: (0, 0)),
            pl.BlockSpec((1, HID), lambda b3, j: (0, 0)),
            pl.BlockSpec((1, HID), lambda b3, j: (0, 0)),
        ],
        out_specs=pl.BlockSpec((1, blk, HID), lambda b3, j: (b3, j, 0)),
        out_shape=jax.ShapeDtypeStruct((BN, NP1, HID), jnp.float32),
    )(aggp, h2, h1, dinv.reshape(1, NP1), b2.reshape(1, HID),
      g.reshape(1, HID), b.reshape(1, HID))


def _tc_s2t_scores(hflat, p):
    m = hflat.shape[0]
    blk = 512
    f1w, f1b = p['fc1_W'], p['fc1_b']
    lng, lnb = p['ln_g'], p['ln_b']
    f2w = p['fc2_W'][:, 0]
    f2b = p['fc2_b'][0]

    def body(h_ref, w1_ref, b1_ref, g_ref, b_ref, w2_ref, b2_ref, o_ref):
        z = jnp.dot(h_ref[...], w1_ref[...],
                    preferred_element_type=jnp.float32) + b1_ref[0][None, :]
        mu = jnp.mean(z, axis=-1, keepdims=True)
        v = jnp.mean((z - mu) ** 2, axis=-1, keepdims=True)
        t = jnp.tanh((z - mu) * lax.rsqrt(v + EPS) * g_ref[0][None, :]
                     + b_ref[0][None, :])
        o_ref[0] = jnp.sum(t * w2_ref[0][None, :], axis=-1) + b2_ref[0, 0]

    return pl.pallas_call(
        body,
        grid=(m // blk,),
        in_specs=[
            pl.BlockSpec((blk, HID), lambda i: (i, 0)),
            pl.BlockSpec((HID, 4 * HID), lambda i: (0, 0)),
            pl.BlockSpec((1, 4 * HID), lambda i: (0, 0)),
            pl.BlockSpec((1, 4 * HID), lambda i: (0, 0)),
            pl.BlockSpec((1, 4 * HID), lambda i: (0, 0)),
            pl.BlockSpec((1, 4 * HID), lambda i: (0, 0)),
            pl.BlockSpec((1, 1), lambda i: (0, 0)),
        ],
        out_specs=pl.BlockSpec((1, blk), lambda i: (0, i)),
        out_shape=jax.ShapeDtypeStruct((1, m), jnp.float32),
    )(hflat, f1w, f1b.reshape(1, -1), lng.reshape(1, -1),
      lnb.reshape(1, -1), f2w.reshape(1, -1),
      p['fc2_b'].reshape(1, 1)).reshape(m)


def _tc_sexp(s, npad, nvalid):
    def body(s_ref, o_ref):
        row = s_ref[0]
        iot = lax.broadcasted_iota(jnp.int32, (npad,), 0)
        msk = iot < nvalid
        sm = jnp.where(msk, row, -1e30)
        cmax = jnp.max(sm)
        o_ref[0] = jnp.where(msk, jnp.exp(row - cmax), 0.0)

    return pl.pallas_call(
        body,
        grid=(BN,),
        in_specs=[pl.BlockSpec((1, npad), lambda b: (0, b))],
        out_specs=pl.BlockSpec((1, npad), lambda b: (0, b)),
        out_shape=jax.ShapeDtypeStruct((1, BN * npad), jnp.float32),
    )(s.reshape(1, BN * npad)).reshape(BN, npad)


def _tc_gatpre(lv2p, d0, d1, gw, a_s, a_d, nvalid):
    def body(l_ref, d0_ref, d1_ref, w_ref, as_ref, ad_ref, hw_ref, hs_ref,
             hd_ref, c_ref, es_ref):
        den = d0_ref[0] + d1_ref[0] + 1e-16
        lv2 = (l_ref[0, 0] + l_ref[0, 1]) / den[:, None]
        hw = jnp.dot(lv2, w_ref[...], preferred_element_type=jnp.float32)
        hs = jnp.sum(hw * as_ref[0][None, :], axis=-1)
        hd = jnp.sum(hw * ad_ref[0][None, :], axis=-1)
        iot = lax.broadcasted_iota(jnp.int32, (NP2,), 0)
        msk = iot < nvalid
        hs = jnp.where(msk, hs, 0.0)
        hd = jnp.where(msk, hd, 0.0)
        cb = jnp.maximum(jnp.max(jnp.where(msk, hs, -1e30))
                         + jnp.max(jnp.where(msk, hd, -1e30)), 0.0)
        e = hs + hd
        e = jnp.where(e >= 0.0, e, 0.2 * e)
        hw_ref[0] = hw
        hs_ref[0] = hs
        hd_ref[0] = hd
        c_ref[0] = jnp.full((NP2,), cb, jnp.float32)
        es_ref[0] = jnp.where(msk, jnp.exp(e - cb), 0.0)

    outs = pl.pallas_call(
        body,
        grid=(BN,),
        in_specs=[
            pl.BlockSpec((1, 2, NP2, HID), lambda b: (b, 0, 0, 0)),
            pl.BlockSpec((1, NP2), lambda b: (0, b)),
            pl.BlockSpec((1, NP2), lambda b: (0, b)),
            pl.BlockSpec((HID, HID), lambda b: (0, 0)),
            pl.BlockSpec((1, HID), lambda b: (0, 0)),
            pl.BlockSpec((1, HID), lambda b: (0, 0)),
        ],
        out_specs=[
            pl.BlockSpec((1, NP2, HID), lambda b: (b, 0, 0)),
            pl.BlockSpec((1, NP2), lambda b: (0, b)),
            pl.BlockSpec((1, NP2), lambda b: (0, b)),
            pl.BlockSpec((1, NP2), lambda b: (0, b)),
            pl.BlockSpec((1, NP2), lambda b: (0, b)),
        ],
        out_shape=[
            jax.ShapeDtypeStruct((BN, NP2, HID), jnp.float32),
            jax.ShapeDtypeStruct((1, BN * NP2), jnp.float32),
            jax.ShapeDtypeStruct((1, BN * NP2), jnp.float32),
            jax.ShapeDtypeStruct((1, BN * NP2), jnp.float32),
            jax.ShapeDtypeStruct((1, BN * NP2), jnp.float32),
        ],
    )(lv2p, d0.reshape(1, BN * NP2), d1.reshape(1, BN * NP2), gw,
      a_s.reshape(1, HID), a_d.reshape(1, HID))
    return outs


def _tc_gatpost(gaggp, hw, ex_self, d0, d1, gb, g, b):
    def body(a_ref, hw_ref, es_ref, d0_ref, d1_ref, gb_ref, g_ref, b_ref,
             o_ref):
        es = es_ref[0]
        den = d0_ref[0] + d1_ref[0] + es + 1e-16
        pre = ((a_ref[0, 0] + a_ref[0, 1] + es[:, None] * hw_ref[0])
               / den[:, None] + gb_ref[0][None, :])
        m = jnp.mean(pre, axis=-1, keepdims=True)
        v = jnp.mean((pre - m) ** 2, axis=-1, keepdims=True)
        o_ref[0] = ((pre - m) * lax.rsqrt(v + EPS) * g_ref[0][None, :]
                    + b_ref[0][None, :])

    return pl.pallas_call(
        body,
        grid=(BN,),
        in_specs=[
            pl.BlockSpec((1, 2, NP2, HID), lambda b2: (b2, 0, 0, 0)),
            pl.BlockSpec((1, NP2, HID), lambda b2: (b2, 0, 0)),
            pl.BlockSpec((1, NP2), lambda b2: (0, b2)),
            pl.BlockSpec((1, NP2), lambda b2: (0, b2)),
            pl.BlockSpec((1, NP2), lambda b2: (0, b2)),
            pl.BlockSpec((1, HID), lambda b2: (0, 0)),
            pl.BlockSpec((1, HID), lambda b2: (0, 0)),
            pl.BlockSpec((1, HID), lambda b2: (0, 0)),
        ],
        out_specs=pl.BlockSpec((1, NP2, HID), lambda b2: (b2, 0, 0)),
        out_shape=jax.ShapeDtypeStruct((BN, NP2, HID), jnp.float32),
    )(gaggp, hw, ex_self.reshape(1, BN * NP2), d0.reshape(1, BN * NP2),
      d1.reshape(1, BN * NP2), gb.reshape(1, HID),
      g.reshape(1, HID), b.reshape(1, HID))


def _tc_final(parts, d0, d1):
    def body(p_ref, d0_ref, d1_ref, o_ref):
        den = d0_ref[0] + d1_ref[0] + 1e-16
        o_ref[0] = (p_ref[0, 0] + p_ref[0, 1]) / den[:, None]

    return pl.pallas_call(
        body,
        grid=(BN,),
        in_specs=[
            pl.BlockSpec((1, 2, NP3, HID), lambda b: (b, 0, 0, 0)),
            pl.BlockSpec((1, NP3), lambda b: (0, b)),
            pl.BlockSpec((1, NP3), lambda b: (0, b)),
        ],
        out_specs=pl.BlockSpec((1, NP3, HID), lambda b: (b, 0, 0)),
        out_shape=jax.ShapeDtypeStruct((BN, NP3, HID), jnp.float32),
    )(parts, d0.reshape(1, BN * NP3), d1.reshape(1, BN * NP3))


def kernel(x, batch_num, level_11_edge_index, level_22_edge_index,
           level_21_seg, level_32_seg, level_13_indicator, params):
    p = params
    B = x.shape[0]

    i32 = jnp.int32
    src1 = level_11_edge_index[0].astype(i32)
    dst1 = level_11_edge_index[1].astype(i32)
    pad1 = jnp.full((EP1 - L1E,), NP1 - 1, i32)
    src1p = jnp.concatenate([src1, pad1])
    dst1p = jnp.concatenate([dst1, pad1])

    src2 = level_22_edge_index[0].astype(i32)
    dst2 = level_22_edge_index[1].astype(i32)
    pad2 = jnp.full((EP2 - L2E,), NP2 - 1, i32)
    src2p = jnp.concatenate([src2, pad2])
    dst2p = jnp.concatenate([dst2, pad2])

    seg21 = level_21_seg.astype(i32)
    seg21p = jnp.concatenate([seg21, jnp.full((SP1 - L1N,), NP2 - 1, i32)])
    iota1p = jnp.minimum(jnp.arange(SP1, dtype=i32), NP1 - 1)

    seg32 = level_32_seg.astype(i32)
    seg32p = jnp.concatenate([seg32, jnp.full((NP2 - L2N,), NP3 - 1, i32)])
    iota2p = jnp.arange(NP2, dtype=i32)

    xp = jnp.pad(x, ((0, 0), (0, NP1 - L1N)))

    degp = _make_sc_deg(EP1, NP1)(dst1p)
    dinv, xs = _tc_dinv(degp, xp)

    alphap = _make_sc_gcn1(EP1, NP1, NP1)(xs, src1p, dst1p)
    h1 = _tc_gcn1ln(alphap.reshape(2, BN, NP1), xp, dinv,
                    p['gcn1_W'][0], p['ln1_g'], p['ln1_b'])

    h1f = h1.reshape(BN * NP1, HID)
    dinv4 = jnp.tile(dinv, BN)
    h2s = _tc_matmul_scale(h1f, p['gcn2_W'], dinv4).reshape(BN, NP1, HID)
    rows1 = _make_sc_rows(EP1, NP1, NP1, False)
    aggp = rows1(h2s[0], h2s[1], h2s[2], h2s[3],
                 src1p, dst1p).reshape(BN, 2, NP1, HID)
    h = _tc_gcn2post(aggp, h2s, h1, dinv, p['gcn2_b'],
                     p['ln2_g'], p['ln2_b'])

    s12 = _tc_s2t_scores(h.reshape(BN * NP1, HID),
                         p['s2t12']).reshape(BN, NP1)
    ex12 = _tc_sexp(s12, NP1, L1N)
    ex12p = jnp.pad(ex12, ((0, 0), (0, SP1 - NP1)))
    den12 = _make_sc_segden(SP1, NP2)(ex12p, seg21p)
    d12 = den12.reshape(2, 4 * NP2)
    hs12 = _tc_scalerows(h, ex12)
    rows12 = _make_sc_rows(SP1, NP1, NP2, False)
    lv2p = rows12(hs12[0], hs12[1], hs12[2], hs12[3],
                  iota1p, seg21p).reshape(BN, 2, NP2, HID)

    hw, hs, hd, cb, ex_self = _tc_gatpre(lv2p, d12[0], d12[1], p['gat_W'],
                                         p['gat_as'], p['gat_ad'], L2N)
    cb64 = cb.reshape(BN, NP2)[:, :16].reshape(64)
    exg = _make_sc_gatex(EP2, NP2)(hs.reshape(4 * NP2), hd.reshape(4 * NP2),
                                   cb64, src2p, dst2p)
    deng = _make_sc_segden(EP2, NP2)(exg, dst2p)
    dg = deng.reshape(2, 4 * NP2)
    rowsg = _make_sc_rows(EP2, NP2, NP2, True)
    gaggp = rowsg(hw[0], hw[1], hw[2], hw[3],
                  src2p, dst2p, exg).reshape(BN, 2, NP2, HID)
    dgb = dg.reshape(2, BN, NP2)
    lv2f = _tc_gatpost(gaggp, hw, ex_self, dgb[0], dgb[1],
                       p['gat_b'], p['lng_g'], p['lng_b'])

    s23 = _tc_s2t_scores(lv2f.reshape(BN * NP2, HID),
                         p['s2t23']).reshape(BN, NP2)
    ex23 = _tc_sexp(s23, NP2, L2N)
    den23 = _make_sc_segden(NP2, NP3)(ex23, seg32p)
    d23 = den23.reshape(2, 4 * NP3)
    hs23 = _tc_scalerows(lv2f, ex23)
    rows23 = _make_sc_rows(NP2, NP2, NP3, False)
    lv3p = rows23(hs23[0], hs23[1], hs23[2], hs23[3],
                  iota2p, seg32p).reshape(BN, 2, NP3, HID)
    lv3 = _tc_final(lv3p, d23[0], d23[1])
    return lv3[:, :L3N, :]


# speedup vs baseline: 10.7057x; 1.0088x over previous
import functools
import math

import jax
import jax.numpy as jnp
from jax import lax
from jax.experimental import pallas as pl
from jax.experimental.pallas import tpu as pltpu
from jax.experimental.pallas import tpu_sc as plsc

L1N = 10000; L2N = 1000; L3N = 50; L1E = 160000; L2E = 16000; HID = 128; BN = 4

NP1 = 10240
NP2 = 1024
NP3 = 1024
EP1 = 163840
EP2 = 16384
SP1 = 12288
NW = 32
NSUB = 16
EPS = 1e-5
_STOP_AT = 99

_mesh = None
_SC_PARAMS = pltpu.CompilerParams(needs_layout_passes=False)


def _get_mesh():
    global _mesh
    if _mesh is None:
        _mesh = plsc.VectorSubcoreMesh(core_axis_name="c", subcore_axis_name="s",
                                       num_cores=2, num_subcores=NSUB)
    return _mesh


def _wid():
    return lax.axis_index("s") * 2 + lax.axis_index("c")


def _zero_vmem(ref, n):
    z = jnp.zeros((16,), jnp.float32)

    def body(i, _):
        ref[pl.ds(i * 16, 16)] = z
        return 0

    lax.fori_loop(0, n // 16, body, 0)


def _zero_vmem2d(ref, rows):
    z = jnp.zeros((16,), jnp.float32)

    def body(i, _):
        for j in range(8):
            ref[i, pl.ds(j * 16, 16)] = z
        return 0

    lax.fori_loop(0, rows, body, 0)


def _blocks(ep):
    ew = ep // NW
    if ew >= 128:
        assert ew % 128 == 0
        return ew, 128, ew // 128
    assert ew % 16 == 0
    return ew, ew, 1


@functools.lru_cache(maxsize=None)
def _make_sc_deg(ep, t):
    ew, bs, nb = _blocks(ep)
    zl = t // NSUB

    @functools.partial(
        pl.kernel,
        out_type=jax.ShapeDtypeStruct((2 * t,), jnp.float32),
        mesh=_get_mesh(),
        compiler_params=_SC_PARAMS,
        scratch_types=[
            pltpu.VMEM((bs,), jnp.int32),
            pltpu.VMEM((bs,), jnp.float32),
            pltpu.VMEM((zl,), jnp.float32),
            pltpu.VMEM_SHARED((t,), jnp.float32),
        ],
    )
    def k(dst_hbm, out_hbm, idx_v, ones_v, zbuf, acc_sh):
        c = lax.axis_index("c")
        s = lax.axis_index("s")
        wid = _wid()
        _zero_vmem(zbuf, zl)
        _zero_vmem(ones_v, bs)
        ones = jnp.ones((16,), jnp.float32)

        def fill(i, _):
            ones_v[pl.ds(i * 16, 16)] = ones
            return 0

        lax.fori_loop(0, bs // 16, fill, 0)
        pltpu.sync_copy(zbuf, acc_sh.at[pl.ds(s * zl, zl)])
        plsc.subcore_barrier()

        def blk(i, _):
            base = wid * ew + i * bs
            pltpu.sync_copy(dst_hbm.at[pl.ds(base, bs)], idx_v)
            pltpu.sync_copy(ones_v, acc_sh.at[idx_v], add=True)
            return 0

        lax.fori_loop(0, nb, blk, 0)
        plsc.subcore_barrier()
        pltpu.sync_copy(acc_sh.at[pl.ds(s * zl, zl)],
                        out_hbm.at[pl.ds(c * t + s * zl, zl)])

    return k


@functools.lru_cache(maxsize=None)
def _make_sc_gcn1(ep, n, t):
    ew, bs, nb = _blocks(ep)
    zl = 4 * t // NSUB

    @functools.partial(
        pl.kernel,
        out_type=jax.ShapeDtypeStruct((2 * 4 * t,), jnp.float32),
        mesh=_get_mesh(),
        compiler_params=_SC_PARAMS,
        scratch_types=[
            pltpu.VMEM((4 * n,), jnp.float32),
            pltpu.VMEM((bs,), jnp.int32),
            pltpu.VMEM((bs,), jnp.int32),
            pltpu.VMEM((bs,), jnp.int32),
            pltpu.VMEM((bs,), jnp.float32),
            pltpu.VMEM((zl,), jnp.float32),
            pltpu.VMEM_SHARED((4 * t,), jnp.float32),
        ],
    )
    def k(x_hbm, src_hbm, dst_hbm, out_hbm,
          xtab, sv, dv, iv, vv, zbuf, acc_sh):
        c = lax.axis_index("c")
        s = lax.axis_index("s")
        wid = _wid()
        pltpu.sync_copy(x_hbm, xtab)
        _zero_vmem(zbuf, zl)
        pltpu.sync_copy(zbuf, acc_sh.at[pl.ds(s * zl, zl)])
        plsc.subcore_barrier()

        def blk(i, _):
            base = wid * ew + i * bs
            pltpu.sync_copy(src_hbm.at[pl.ds(base, bs)], sv)
            pltpu.sync_copy(dst_hbm.at[pl.ds(base, bs)], dv)
            for b in range(4):
                def vec(j, _):
                    svj = sv[pl.ds(j * 16, 16)] + (b * n)
                    vv[pl.ds(j * 16, 16)] = plsc.load_gather(xtab, [svj])
                    iv[pl.ds(j * 16, 16)] = dv[pl.ds(j * 16, 16)] + (b * t)
                    return 0

                lax.fori_loop(0, bs // 16, vec, 0)
                pltpu.sync_copy(vv, acc_sh.at[iv], add=True)
            return 0

        lax.fori_loop(0, nb, blk, 0)
        plsc.subcore_barrier()
        pltpu.sync_copy(acc_sh.at[pl.ds(s * zl, zl)],
                        out_hbm.at[pl.ds(c * 4 * t + s * zl, zl)])

    return k


@functools.lru_cache(maxsize=None)
def _make_sc_segden(ep, t):
    ew, bs, nb = _blocks(ep)
    zl = 4 * t // NSUB
    small = zl < 256
    if small:
        zl = 4 * t

    @functools.partial(
        pl.kernel,
        out_type=jax.ShapeDtypeStruct((2 * 4 * t,), jnp.float32),
        mesh=_get_mesh(),
        compiler_params=_SC_PARAMS,
        scratch_types=[
            pltpu.VMEM((bs,), jnp.int32),
            pltpu.VMEM((bs,), jnp.int32),
            pltpu.VMEM((bs,), jnp.float32),
            pltpu.VMEM((zl,), jnp.float32),
            pltpu.VMEM_SHARED((4 * t,), jnp.float32),
        ],
    )
    def k(vals_hbm, dst_hbm, out_hbm, dv, iv, vv, zbuf, acc_sh):
        c = lax.axis_index("c")
        s = lax.axis_index("s")
        wid = _wid()
        _zero_vmem(zbuf, zl)
        if small:
            @pl.when(s == 0)
            def _():
                pltpu.sync_copy(zbuf, acc_sh)
        else:
            pltpu.sync_copy(zbuf, acc_sh.at[pl.ds(s * zl, zl)])
        plsc.subcore_barrier()

        def blk(i, _):
            base = wid * ew + i * bs
            pltpu.sync_copy(dst_hbm.at[pl.ds(base, bs)], dv)
            for b in range(4):
                pltpu.sync_copy(vals_hbm.at[b, pl.ds(base, bs)], vv)

                def vec(j, _):
                    iv[pl.ds(j * 16, 16)] = dv[pl.ds(j * 16, 16)] + (b * t)
                    return 0

                lax.fori_loop(0, bs // 16, vec, 0)
                pltpu.sync_copy(vv, acc_sh.at[iv], add=True)
            return 0

        lax.fori_loop(0, nb, blk, 0)
        plsc.subcore_barrier()
        if small:
            @pl.when(s == 0)
            def _():
                pltpu.sync_copy(acc_sh, out_hbm.at[pl.ds(c * 4 * t, 4 * t)])
        else:
            pltpu.sync_copy(acc_sh.at[pl.ds(s * zl, zl)],
                            out_hbm.at[pl.ds(c * 4 * t + s * zl, zl)])

    return k


@functools.lru_cache(maxsize=None)
def _make_sc_gatex(ep, t):
    ew, bs, nb = _blocks(ep)

    @functools.partial(
        pl.kernel,
        out_type=jax.ShapeDtypeStruct((4, ep), jnp.float32),
        mesh=_get_mesh(),
        compiler_params=_SC_PARAMS,
        scratch_types=[
            pltpu.VMEM((4 * t,), jnp.float32),
            pltpu.VMEM((4 * t,), jnp.float32),
            pltpu.VMEM((64,), jnp.float32),
            pltpu.VMEM((bs,), jnp.int32),
            pltpu.VMEM((bs,), jnp.int32),
            pltpu.VMEM((bs,), jnp.float32),
        ],
    )
    def k(hs_hbm, hd_hbm, c_hbm, src_hbm, dst_hbm, out_hbm,
          ts, td, tc, sv, dv, ov):
        wid = _wid()
        pltpu.sync_copy(hs_hbm, ts)
        pltpu.sync_copy(hd_hbm, td)
        pltpu.sync_copy(c_hbm, tc)

        def blk(i, _):
            base = wid * ew + i * bs
            pltpu.sync_copy(src_hbm.at[pl.ds(base, bs)], sv)
            pltpu.sync_copy(dst_hbm.at[pl.ds(base, bs)], dv)
            for b in range(4):
                cb = tc[pl.ds(b * 16, 16)]

                def vec(j, _):
                    sl = pl.ds(j * 16, 16)
                    a = plsc.load_gather(ts, [sv[sl] + (b * t)])
                    d = plsc.load_gather(td, [dv[sl] + (b * t)])
                    e = a + d
                    e = jnp.where(e >= 0.0, e, 0.2 * e)
                    ov[sl] = jnp.exp(e - cb)
                    return 0

                lax.fori_loop(0, bs // 16, vec, 0)
                pltpu.sync_copy(ov, out_hbm.at[b, pl.ds(base, bs)])
            return 0

        lax.fori_loop(0, nb, blk, 0)

    return k


@functools.lru_cache(maxsize=None)
def _make_sc_rows(ep, n, mp, scaled):
    ew, bs, nb = _blocks(ep)
    rpt = mp // NSUB

    scratch = [
        pltpu.VMEM((bs,), jnp.int32),
        pltpu.VMEM((bs,), jnp.int32),
        pltpu.VMEM((bs,), jnp.int32),
        pltpu.VMEM((bs,), jnp.int32),
        pltpu.VMEM((bs,), jnp.float32),
        pltpu.VMEM((bs, 128), jnp.float32),
        pltpu.VMEM((bs, 128), jnp.float32),
        pltpu.VMEM_SHARED((mp, 128), jnp.float32),
        pltpu.SemaphoreType.DMA,
        pltpu.SemaphoreType.DMA,
        pltpu.SemaphoreType.DMA,
        pltpu.SemaphoreType.DMA,
    ]

    def body(tabs, src_hbm, dst_hbm, w_hbm, out_hbm,
             sv0, sv1, dv0, dv1, wv, r0, r1, acc_sh,
             gsem0, gsem1, ssem0, ssem1):
        c = lax.axis_index("c")
        s = lax.axis_index("s")
        wid = _wid()
        svs, dvs, rs = (sv0, sv1), (dv0, dv1), (r0, r1)
        gsems, ssems = (gsem0, gsem1), (ssem0, ssem1)
        zb = min(bs, 128)
        nf, zt = rpt // zb, rpt % zb
        nco = (rpt + 127) // 128

        for bnum in range(4):
            tab = tabs[bnum]
            _zero_vmem2d(r0, zb)
            for tzi in range(nf):
                pltpu.sync_copy(r0.at[pl.ds(0, zb)],
                                acc_sh.at[pl.ds(s * rpt + tzi * zb, zb)])
            if zt:
                pltpu.sync_copy(r0.at[pl.ds(0, zt)],
                                acc_sh.at[pl.ds(s * rpt + nf * zb, zt)])
            plsc.subcore_barrier()

            pltpu.sync_copy(src_hbm.at[pl.ds(wid * ew, bs)], sv0)
            gd = [pltpu.async_copy(tab.at[sv0], r0, gsem0), None]
            sd = [None, None]
            for i in range(nb):
                cur = i & 1
                nxt = 1 - cur
                if i + 1 < nb:
                    nbase = wid * ew + (i + 1) * bs
                    pltpu.sync_copy(src_hbm.at[pl.ds(nbase, bs)], svs[nxt])
                    if sd[nxt] is not None:
                        sd[nxt].wait()
                        sd[nxt] = None
                    gd[nxt] = pltpu.async_copy(tab.at[svs[nxt]],
                                               rs[nxt], gsems[nxt])
                base = wid * ew + i * bs
                pltpu.sync_copy(dst_hbm.at[pl.ds(base, bs)], dvs[cur])
                if scaled:
                    pltpu.sync_copy(w_hbm.at[bnum, pl.ds(base, bs)], wv)
                gd[cur].wait()
                if scaled:
                    def scale(r, _, _rs=rs[cur]):
                        wr = plsc.load_gather(
                            wv, [jnp.full((16,), r, jnp.int32)])
                        for j in range(8):
                            sl = pl.ds(j * 16, 16)
                            _rs[r, sl] = _rs[r, sl] * wr
                        return 0
                    lax.fori_loop(0, bs, scale, 0)
                sd[cur] = pltpu.async_copy(rs[cur], acc_sh.at[dvs[cur]],
                                           ssems[cur], add=True)
            for d in sd:
                if d is not None:
                    d.wait()
            plsc.subcore_barrier()

            base_o = (2 * bnum + c) * mp + s * rpt
            for t2 in range(nco):
                sz = min(128, rpt - t2 * 128)
                pltpu.sync_copy(acc_sh.at[pl.ds(s * rpt + t2 * 128, sz)],
                                out_hbm.at[pl.ds(base_o + t2 * 128, sz)])

    if scaled:
        def body_s(t0, t1, t2, t3, src_hbm, dst_hbm, w_hbm, out_hbm, *scr):
            body((t0, t1, t2, t3), src_hbm, dst_hbm, w_hbm, out_hbm, *scr)
        fn = body_s
    else:
        def body_n(t0, t1, t2, t3, src_hbm, dst_hbm, out_hbm, *scr):
            body((t0, t1, t2, t3), src_hbm, dst_hbm, None, out_hbm, *scr)
        fn = body_n

    return pl.kernel(
        fn,
        out_type=jax.ShapeDtypeStruct((4 * 2 * mp, 128), jnp.float32),
        mesh=_get_mesh(),
        compiler_params=_SC_PARAMS,
        scratch_types=scratch,
    )


def _tc_dinv(degp, xp):
    def body(d_ref, x_ref, o_ref, xs_ref):
        d = d_ref[0] + d_ref[1] + 1.0
        di = lax.rsqrt(d.reshape(80, 128))
        o_ref[...] = di
        dflat = di.reshape(1, NP1)
        xs_ref[...] = (x_ref[...].reshape(BN, NP1)
                       * dflat).reshape(1, BN * NP1)

    dinv, xs = pl.pallas_call(
        body,
        out_shape=[jax.ShapeDtypeStruct((80, 128), jnp.float32),
                   jax.ShapeDtypeStruct((1, BN * NP1), jnp.float32)],
    )(degp.reshape(2, 80, 128), xp.reshape(1, BN * NP1))
    return dinv.reshape(NP1), xs.reshape(BN * NP1)


def _tc_gcn1ln(alphap, x, dinv, w1, g, b):
    blk = 512
    nj = NP1 // blk

    def body(a_ref, x_ref, di_ref, w_ref, g_ref, b_ref, o_ref):
        di = di_ref[0]
        alpha = di * (a_ref[0] + a_ref[1] + di * x_ref[0])
        w = w_ref[0]
        mw = jnp.mean(w)
        vw = jnp.mean((w - mw) ** 2)
        wc = w - mw
        denom = lax.rsqrt(alpha * alpha * vw + EPS)
        o_ref[0] = ((alpha * denom)[:, None] * wc[None, :] * g_ref[0][None, :]
                    + b_ref[0][None, :])

    grid = (BN, nj)
    return pl.pallas_call(
        body,
        grid=grid,
        in_specs=[
            pl.BlockSpec((2, blk), lambda b2, j: (0, b2 * nj + j)),
            pl.BlockSpec((1, blk), lambda b2, j: (0, b2 * nj + j)),
            pl.BlockSpec((1, blk), lambda b2, j: (0, j)),
            pl.BlockSpec((1, HID), lambda b2, j: (0, 0)),
            pl.BlockSpec((1, HID), lambda b2, j: (0, 0)),
            pl.BlockSpec((1, HID), lambda b2, j: (0, 0)),
        ],
        out_specs=pl.BlockSpec((1, blk, HID), lambda b2, j: (b2, j, 0)),
        out_shape=jax.ShapeDtypeStruct((BN, NP1, HID), jnp.float32),
    )(alphap.reshape(2, BN * NP1), x.reshape(1, BN * NP1),
      dinv.reshape(1, NP1), w1.reshape(1, HID),
      g.reshape(1, HID), b.reshape(1, HID))


def _tc_matmul(h, w):
    m, kdim = h.shape
    kout = w.shape[1]
    blk = 512

    def body(h_ref, w_ref, o_ref):
        o_ref[...] = jnp.dot(h_ref[...], w_ref[...],
                             preferred_element_type=jnp.float32)

    return pl.pallas_call(
        body,
        grid=(m // blk,),
        in_specs=[
            pl.BlockSpec((blk, kdim), lambda i: (i, 0)),
            pl.BlockSpec((kdim, kout), lambda i: (0, 0)),
        ],
        out_specs=pl.BlockSpec((blk, kout), lambda i: (i, 0)),
        out_shape=jax.ShapeDtypeStruct((m, kout), jnp.float32),
    )(h, w)


def _tc_matmul_scale(h, w, sc):
    m, kdim = h.shape
    kout = w.shape[1]
    blk = 512

    def body(h_ref, w_ref, s_ref, o_ref):
        o_ref[...] = jnp.dot(h_ref[...], w_ref[...],
                             preferred_element_type=jnp.float32) \
            * s_ref[0][:, None]

    return pl.pallas_call(
        body,
        grid=(m // blk,),
        in_specs=[
            pl.BlockSpec((blk, kdim), lambda i: (i, 0)),
            pl.BlockSpec((kdim, kout), lambda i: (0, 0)),
            pl.BlockSpec((1, blk), lambda i: (0, i)),
        ],
        out_specs=pl.BlockSpec((blk, kout), lambda i: (i, 0)),
        out_shape=jax.ShapeDtypeStruct((m, kout), jnp.float32),
    )(h, w, sc.reshape(1, m))


def _tc_scalerows(h, ex):
    bdim, ndim = h.shape[0], h.shape[1]
    blk = 512
    nj = ndim // blk

    def body(h_ref, e_ref, o_ref):
        o_ref[0] = h_ref[0] * e_ref[0][:, None]

    return pl.pallas_call(
        body,
        grid=(bdim, nj),
        in_specs=[
            pl.BlockSpec((1, blk, HID), lambda b, j: (b, j, 0)),
            pl.BlockSpec((1, blk), lambda b, j: (0, b * nj + j)),
        ],
        out_specs=pl.BlockSpec((1, blk, HID), lambda b, j: (b, j, 0)),
        out_shape=jax.ShapeDtypeStruct((bdim, ndim, HID), jnp.float32),
    )(h, ex.reshape(1, bdim * ndim))


def _tc_gcn2post(aggp, h2, h1, dinv, b2, g, b):
    blk = 512

    def body(a_ref, h2_ref, h1_ref, di_ref, b2_ref, g_ref, b_ref, o_ref):
        pre = (di_ref[0][:, None] * (a_ref[0, 0] + a_ref[0, 1] + h2_ref[0])
               + b2_ref[0][None, :])
        m = jnp.mean(pre, axis=-1, keepdims=True)
        v = jnp.mean((pre - m) ** 2, axis=-1, keepdims=True)
        o_ref[0] = h1_ref[0] + ((pre - m) * lax.rsqrt(v + EPS)
                                * g_ref[0][None, :] + b_ref[0][None, :])

    grid = (BN, NP1 // blk)
    return pl.pallas_call(
        body,
        grid=grid,
        in_specs=[
            pl.BlockSpec((1, 2, blk, HID), lambda b3, j: (b3, 0, j, 0)),
            pl.BlockSpec((1, blk, HID), lambda b3, j: (b3, j, 0)),
            pl.BlockSpec((1, blk, HID), lambda b3, j: (b3, j, 0)),
            pl.BlockSpec((1, blk), lambda b3, j: (0, j)),
            pl.BlockSpec((1, HID), lambda b3, j: (0, 0)),
            pl.BlockSpec((1, HID), lambda b3, j: (0, 0)),
            pl.BlockSpec((1, HID), lambda b3, j: (0, 0)),
        ],
        out_specs=pl.BlockSpec((1, blk, HID), lambda b3, j: (b3, j, 0)),
        out_shape=jax.ShapeDtypeStruct((BN, NP1, HID), jnp.float32),
    )(aggp, h2, h1, dinv.reshape(1, NP1), b2.reshape(1, HID),
      g.reshape(1, HID), b.reshape(1, HID))


def _tc_s2t_scores(hflat, p):
    m = hflat.shape[0]
    blk = 512
    f1w, f1b = p['fc1_W'], p['fc1_b']
    lng, lnb = p['ln_g'], p['ln_b']
    f2w = p['fc2_W'][:, 0]
    f2b = p['fc2_b'][0]

    def body(h_ref, w1_ref, b1_ref, g_ref, b_ref, w2_ref, b2_ref, o_ref):
        z = jnp.dot(h_ref[...], w1_ref[...],
                    preferred_element_type=jnp.float32) + b1_ref[0][None, :]
        mu = jnp.mean(z, axis=-1, keepdims=True)
        v = jnp.mean((z - mu) ** 2, axis=-1, keepdims=True)
        t = jnp.tanh((z - mu) * lax.rsqrt(v + EPS) * g_ref[0][None, :]
                     + b_ref[0][None, :])
        o_ref[0] = jnp.sum(t * w2_ref[0][None, :], axis=-1) + b2_ref[0, 0]

    return pl.pallas_call(
        body,
        grid=(m // blk,),
        in_specs=[
            pl.BlockSpec((blk, HID), lambda i: (i, 0)),
            pl.BlockSpec((HID, 4 * HID), lambda i: (0, 0)),
            pl.BlockSpec((1, 4 * HID), lambda i: (0, 0)),
            pl.BlockSpec((1, 4 * HID), lambda i: (0, 0)),
            pl.BlockSpec((1, 4 * HID), lambda i: (0, 0)),
            pl.BlockSpec((1, 4 * HID), lambda i: (0, 0)),
            pl.BlockSpec((1, 1), lambda i: (0, 0)),
        ],
        out_specs=pl.BlockSpec((1, blk), lambda i: (0, i)),
        out_shape=jax.ShapeDtypeStruct((1, m), jnp.float32),
    )(hflat, f1w, f1b.reshape(1, -1), lng.reshape(1, -1),
      lnb.reshape(1, -1), f2w.reshape(1, -1),
      p['fc2_b'].reshape(1, 1)).reshape(m)


def _tc_sexp(s, npad, nvalid):
    def body(s_ref, o_ref):
        row = s_ref[0]
        iot = lax.broadcasted_iota(jnp.int32, (npad,), 0)
        msk = iot < nvalid
        sm = jnp.where(msk, row, -1e30)
        cmax = jnp.max(sm)
        o_ref[0] = jnp.where(msk, jnp.exp(row - cmax), 0.0)

    return pl.pallas_call(
        body,
        grid=(BN,),
        in_specs=[pl.BlockSpec((1, npad), lambda b: (0, b))],
        out_specs=pl.BlockSpec((1, npad), lambda b: (0, b)),
        out_shape=jax.ShapeDtypeStruct((1, BN * npad), jnp.float32),
    )(s.reshape(1, BN * npad)).reshape(BN, npad)


def _tc_gatpre(lv2p, d0, d1, gw, a_s, a_d, nvalid):
    def body(l_ref, d0_ref, d1_ref, w_ref, as_ref, ad_ref, hw_ref, hs_ref,
             hd_ref, c_ref, es_ref):
        den = d0_ref[0] + d1_ref[0] + 1e-16
        lv2 = (l_ref[0, 0] + l_ref[0, 1]) / den[:, None]
        hw = jnp.dot(lv2, w_ref[...], preferred_element_type=jnp.float32)
        hs = jnp.sum(hw * as_ref[0][None, :], axis=-1)
        hd = jnp.sum(hw * ad_ref[0][None, :], axis=-1)
        iot = lax.broadcasted_iota(jnp.int32, (NP2,), 0)
        msk = iot < nvalid
        hs = jnp.where(msk, hs, 0.0)
        hd = jnp.where(msk, hd, 0.0)
        cb = jnp.maximum(jnp.max(jnp.where(msk, hs, -1e30))
                         + jnp.max(jnp.where(msk, hd, -1e30)), 0.0)
        e = hs + hd
        e = jnp.where(e >= 0.0, e, 0.2 * e)
        hw_ref[0] = hw
        hs_ref[0] = hs
        hd_ref[0] = hd
        c_ref[0] = jnp.full((NP2,), cb, jnp.float32)
        es_ref[0] = jnp.where(msk, jnp.exp(e - cb), 0.0)

    outs = pl.pallas_call(
        body,
        grid=(BN,),
        in_specs=[
            pl.BlockSpec((1, 2, NP2, HID), lambda b: (b, 0, 0, 0)),
            pl.BlockSpec((1, NP2), lambda b: (0, b)),
            pl.BlockSpec((1, NP2), lambda b: (0, b)),
            pl.BlockSpec((HID, HID), lambda b: (0, 0)),
            pl.BlockSpec((1, HID), lambda b: (0, 0)),
            pl.BlockSpec((1, HID), lambda b: (0, 0)),
        ],
        out_specs=[
            pl.BlockSpec((1, NP2, HID), lambda b: (b, 0, 0)),
            pl.BlockSpec((1, NP2), lambda b: (0, b)),
            pl.BlockSpec((1, NP2), lambda b: (0, b)),
            pl.BlockSpec((1, NP2), lambda b: (0, b)),
            pl.BlockSpec((1, NP2), lambda b: (0, b)),
        ],
        out_shape=[
            jax.ShapeDtypeStruct((BN, NP2, HID), jnp.float32),
            jax.ShapeDtypeStruct((1, BN * NP2), jnp.float32),
            jax.ShapeDtypeStruct((1, BN * NP2), jnp.float32),
            jax.ShapeDtypeStruct((1, BN * NP2), jnp.float32),
            jax.ShapeDtypeStruct((1, BN * NP2), jnp.float32),
        ],
    )(lv2p, d0.reshape(1, BN * NP2), d1.reshape(1, BN * NP2), gw,
      a_s.reshape(1, HID), a_d.reshape(1, HID))
    return outs


def _tc_gatpost(gaggp, hw, ex_self, d0, d1, gb, g, b):
    def body(a_ref, hw_ref, es_ref, d0_ref, d1_ref, gb_ref, g_ref, b_ref,
             o_ref):
        es = es_ref[0]
        den = d0_ref[0] + d1_ref[0] + es + 1e-16
        pre = ((a_ref[0, 0] + a_ref[0, 1] + es[:, None] * hw_ref[0])
               / den[:, None] + gb_ref[0][None, :])
        m = jnp.mean(pre, axis=-1, keepdims=True)
        v = jnp.mean((pre - m) ** 2, axis=-1, keepdims=True)
        o_ref[0] = ((pre - m) * lax.rsqrt(v + EPS) * g_ref[0][None, :]
                    + b_ref[0][None, :])

    return pl.pallas_call(
        body,
        grid=(BN,),
        in_specs=[
            pl.BlockSpec((1, 2, NP2, HID), lambda b2: (b2, 0, 0, 0)),
            pl.BlockSpec((1, NP2, HID), lambda b2: (b2, 0, 0)),
            pl.BlockSpec((1, NP2), lambda b2: (0, b2)),
            pl.BlockSpec((1, NP2), lambda b2: (0, b2)),
            pl.BlockSpec((1, NP2), lambda b2: (0, b2)),
            pl.BlockSpec((1, HID), lambda b2: (0, 0)),
            pl.BlockSpec((1, HID), lambda b2: (0, 0)),
            pl.BlockSpec((1, HID), lambda b2: (0, 0)),
        ],
        out_specs=pl.BlockSpec((1, NP2, HID), lambda b2: (b2, 0, 0)),
        out_shape=jax.ShapeDtypeStruct((BN, NP2, HID), jnp.float32),
    )(gaggp, hw, ex_self.reshape(1, BN * NP2), d0.reshape(1, BN * NP2),
      d1.reshape(1, BN * NP2), gb.reshape(1, HID),
      g.reshape(1, HID), b.reshape(1, HID))


def _tc_final(parts, d0, d1):
    def body(p_ref, d0_ref, d1_ref, o_ref):
        den = d0_ref[0] + d1_ref[0] + 1e-16
        o_ref[0] = (p_ref[0, 0] + p_ref[0, 1]) / den[:, None]

    return pl.pallas_call(
        body,
        grid=(BN,),
        in_specs=[
            pl.BlockSpec((1, 2, NP3, HID), lambda b: (b, 0, 0, 0)),
            pl.BlockSpec((1, NP3), lambda b: (0, b)),
            pl.BlockSpec((1, NP3), lambda b: (0, b)),
        ],
        out_specs=pl.BlockSpec((1, NP3, HID), lambda b: (b, 0, 0)),
        out_shape=jax.ShapeDtypeStruct((BN, NP3, HID), jnp.float32),
    )(parts, d0.reshape(1, BN * NP3), d1.reshape(1, BN * NP3))


def kernel(x, batch_num, level_11_edge_index, level_22_edge_index,
           level_21_seg, level_32_seg, level_13_indicator, params):
    p = params
    B = x.shape[0]

    i32 = jnp.int32
    src1 = level_11_edge_index[0].astype(i32)
    dst1 = level_11_edge_index[1].astype(i32)
    pad1 = jnp.full((EP1 - L1E,), NP1 - 1, i32)
    src1p = jnp.concatenate([src1, pad1])
    dst1p = jnp.concatenate([dst1, pad1])

    src2 = level_22_edge_index[0].astype(i32)
    dst2 = level_22_edge_index[1].astype(i32)
    pad2 = jnp.full((EP2 - L2E,), NP2 - 1, i32)
    src2p = jnp.concatenate([src2, pad2])
    dst2p = jnp.concatenate([dst2, pad2])

    seg21 = level_21_seg.astype(i32)
    seg21p = jnp.concatenate([seg21, jnp.full((SP1 - L1N,), NP2 - 1, i32)])
    iota1p = jnp.minimum(jnp.arange(SP1, dtype=i32), NP1 - 1)

    seg32 = level_32_seg.astype(i32)
    seg32p = jnp.concatenate([seg32, jnp.full((NP2 - L2N,), NP3 - 1, i32)])
    iota2p = jnp.arange(NP2, dtype=i32)

    xp = jnp.pad(x, ((0, 0), (0, NP1 - L1N)))

    degp = _make_sc_deg(EP1, NP1)(dst1p)
    dinv, xs = _tc_dinv(degp, xp)

    alphap = _make_sc_gcn1(EP1, NP1, NP1)(xs, src1p, dst1p)
    h1 = _tc_gcn1ln(alphap.reshape(2, BN, NP1), xp, dinv,
                    p['gcn1_W'][0], p['ln1_g'], p['ln1_b'])

    h1f = h1.reshape(BN * NP1, HID)
    dinv4 = jnp.tile(dinv, BN)
    h2s = _tc_matmul_scale(h1f, p['gcn2_W'], dinv4).reshape(BN, NP1, HID)
    rows1 = _make_sc_rows(EP1, NP1, NP1, False)
    aggp = rows1(h2s[0], h2s[1], h2s[2], h2s[3],
                 src1p, dst1p).reshape(BN, 2, NP1, HID)
    h = _tc_gcn2post(aggp, h2s, h1, dinv, p['gcn2_b'],
                     p['ln2_g'], p['ln2_b'])

    s12 = _tc_s2t_scores(h.reshape(BN * NP1, HID),
                         p['s2t12']).reshape(BN, NP1)
    ex12 = _tc_sexp(s12, NP1, L1N)
    ex12p = jnp.pad(ex12, ((0, 0), (0, SP1 - NP1)))
    den12 = _make_sc_segden(SP1, NP2)(ex12p, seg21p)
    d12 = den12.reshape(2, 4 * NP2)
    hs12 = _tc_scalerows(h, ex12)
    rows12 = _make_sc_rows(SP1, NP1, NP2, False)
    lv2p = rows12(hs12[0], hs12[1], hs12[2], hs12[3],
                  iota1p, seg21p).reshape(BN, 2, NP2, HID)

    hw, hs, hd, cb, ex_self = _tc_gatpre(lv2p, d12[0], d12[1], p['gat_W'],
                                         p['gat_as'], p['gat_ad'], L2N)
    cb64 = cb.reshape(BN, NP2)[:, :16].reshape(64)
    exg = _make_sc_gatex(EP2, NP2)(hs.reshape(4 * NP2), hd.reshape(4 * NP2),
                                   cb64, src2p, dst2p)
    deng = _make_sc_segden(EP2, NP2)(exg, dst2p)
    dg = deng.reshape(2, 4 * NP2)
    rowsg = _make_sc_rows(EP2, NP2, NP2, True)
    gaggp = rowsg(hw[0], hw[1], hw[2], hw[3],
                  src2p, dst2p, exg).reshape(BN, 2, NP2, HID)
    dgb = dg.reshape(2, BN, NP2)
    lv2f = _tc_gatpost(gaggp, hw, ex_self, dgb[0], dgb[1],
                       p['gat_b'], p['lng_g'], p['lng_b'])

    s23 = _tc_s2t_scores(lv2f.reshape(BN * NP2, HID),
                         p['s2t23']).reshape(BN, NP2)
    ex23 = _tc_sexp(s23, NP2, L2N)
    den23 = _make_sc_segden(NP2, NP3)(ex23, seg32p)
    d23 = den23.reshape(2, 4 * NP3)
    hs23 = _tc_scalerows(lv2f, ex23)
    rows23 = _make_sc_rows(NP2, NP2, NP3, False)
    lv3p = rows23(hs23[0], hs23[1], hs23[2], hs23[3],
                  iota2p, seg32p).reshape(BN, 2, NP3, HID)
    lv3 = _tc_final(lv3p, d23[0], d23[1])
    return lv3[:, :L3N, :]


# speedup vs baseline: 10.7170x; 1.0011x over previous
import functools

import jax
import jax.numpy as jnp
from jax import lax
from jax.experimental import pallas as pl
from jax.experimental.pallas import tpu as pltpu
from jax.experimental.pallas import tpu_sc as plsc

L1N = 10000; L2N = 1000; L3N = 50; L1E = 160000; L2E = 16000; HID = 128; BN = 4

NP1 = 10240
NP2 = 1024
NP3 = 1024
EP1 = 163840
EP2 = 16384
SP1 = 12288
NW = 32
NSUB = 16
EPS = 1e-5

_mesh = None
_SC_PARAMS = pltpu.CompilerParams(needs_layout_passes=False)


def _get_mesh():
    global _mesh
    if _mesh is None:
        _mesh = plsc.VectorSubcoreMesh(core_axis_name="c", subcore_axis_name="s",
                                       num_cores=2, num_subcores=NSUB)
    return _mesh


def _wid():
    return lax.axis_index("s") * 2 + lax.axis_index("c")


def _zero_vmem(ref, n):
    z = jnp.zeros((16,), jnp.float32)

    def body(i, _):
        ref[pl.ds(i * 16, 16)] = z
        return 0

    lax.fori_loop(0, n // 16, body, 0)


def _zero_vmem2d(ref, rows):
    z = jnp.zeros((16,), jnp.float32)

    def body(i, _):
        for j in range(8):
            ref[i, pl.ds(j * 16, 16)] = z
        return 0

    lax.fori_loop(0, rows, body, 0)


def _blocks(ep):
    ew = ep // NW
    if ew >= 128:
        assert ew % 128 == 0
        return ew, 128, ew // 128
    assert ew % 16 == 0
    return ew, ew, 1


@functools.lru_cache(maxsize=None)
def _make_sc_deg(ep, t):
    ew, bs, nb = _blocks(ep)
    zl = t // NSUB

    @functools.partial(
        pl.kernel,
        out_type=jax.ShapeDtypeStruct((2 * t,), jnp.float32),
        mesh=_get_mesh(),
        compiler_params=_SC_PARAMS,
        scratch_types=[
            pltpu.VMEM((bs,), jnp.int32),
            pltpu.VMEM((bs,), jnp.float32),
            pltpu.VMEM((zl,), jnp.float32),
            pltpu.VMEM_SHARED((t,), jnp.float32),
        ],
    )
    def k(dst_hbm, out_hbm, idx_v, ones_v, zbuf, acc_sh):
        c = lax.axis_index("c")
        s = lax.axis_index("s")
        wid = _wid()
        _zero_vmem(zbuf, zl)
        _zero_vmem(ones_v, bs)
        ones = jnp.ones((16,), jnp.float32)

        def fill(i, _):
            ones_v[pl.ds(i * 16, 16)] = ones
            return 0

        lax.fori_loop(0, bs // 16, fill, 0)
        pltpu.sync_copy(zbuf, acc_sh.at[pl.ds(s * zl, zl)])
        plsc.subcore_barrier()

        def blk(i, _):
            base = wid * ew + i * bs
            pltpu.sync_copy(dst_hbm.at[pl.ds(base, bs)], idx_v)
            pltpu.sync_copy(ones_v, acc_sh.at[idx_v], add=True)
            return 0

        lax.fori_loop(0, nb, blk, 0)
        plsc.subcore_barrier()
        pltpu.sync_copy(acc_sh.at[pl.ds(s * zl, zl)],
                        out_hbm.at[pl.ds(c * t + s * zl, zl)])

    return k


@functools.lru_cache(maxsize=None)
def _make_sc_gcn1(ep, n, t):
    ew, bs, nb = _blocks(ep)
    zl = 4 * t // NSUB

    @functools.partial(
        pl.kernel,
        out_type=jax.ShapeDtypeStruct((2 * 4 * t,), jnp.float32),
        mesh=_get_mesh(),
        compiler_params=_SC_PARAMS,
        scratch_types=[
            pltpu.VMEM((4 * n,), jnp.float32),
            pltpu.VMEM((bs,), jnp.int32),
            pltpu.VMEM((bs,), jnp.int32),
            pltpu.VMEM((bs,), jnp.int32),
            pltpu.VMEM((bs,), jnp.float32),
            pltpu.VMEM((zl,), jnp.float32),
            pltpu.VMEM_SHARED((4 * t,), jnp.float32),
        ],
    )
    def k(x_hbm, src_hbm, dst_hbm, out_hbm,
          xtab, sv, dv, iv, vv, zbuf, acc_sh):
        c = lax.axis_index("c")
        s = lax.axis_index("s")
        wid = _wid()
        pltpu.sync_copy(x_hbm, xtab)
        _zero_vmem(zbuf, zl)
        pltpu.sync_copy(zbuf, acc_sh.at[pl.ds(s * zl, zl)])
        plsc.subcore_barrier()

        def blk(i, _):
            base = wid * ew + i * bs
            pltpu.sync_copy(src_hbm.at[pl.ds(base, bs)], sv)
            pltpu.sync_copy(dst_hbm.at[pl.ds(base, bs)], dv)
            for b in range(4):
                def vec(j, _):
                    svj = sv[pl.ds(j * 16, 16)] + (b * n)
                    vv[pl.ds(j * 16, 16)] = plsc.load_gather(xtab, [svj])
                    iv[pl.ds(j * 16, 16)] = dv[pl.ds(j * 16, 16)] + (b * t)
                    return 0

                lax.fori_loop(0, bs // 16, vec, 0)
                pltpu.sync_copy(vv, acc_sh.at[iv], add=True)
            return 0

        lax.fori_loop(0, nb, blk, 0)
        plsc.subcore_barrier()
        pltpu.sync_copy(acc_sh.at[pl.ds(s * zl, zl)],
                        out_hbm.at[pl.ds(c * 4 * t + s * zl, zl)])

    return k


@functools.lru_cache(maxsize=None)
def _make_sc_segden(ep, t):
    ew, bs, nb = _blocks(ep)
    zl = 4 * t // NSUB
    small = zl < 256
    if small:
        zl = 4 * t

    @functools.partial(
        pl.kernel,
        out_type=jax.ShapeDtypeStruct((2 * 4 * t,), jnp.float32),
        mesh=_get_mesh(),
        compiler_params=_SC_PARAMS,
        scratch_types=[
            pltpu.VMEM((bs,), jnp.int32),
            pltpu.VMEM((bs,), jnp.int32),
            pltpu.VMEM((bs,), jnp.float32),
            pltpu.VMEM((zl,), jnp.float32),
            pltpu.VMEM_SHARED((4 * t,), jnp.float32),
        ],
    )
    def k(vals_hbm, dst_hbm, out_hbm, dv, iv, vv, zbuf, acc_sh):
        c = lax.axis_index("c")
        s = lax.axis_index("s")
        wid = _wid()
        _zero_vmem(zbuf, zl)
        if small:
            @pl.when(s == 0)
            def _():
                pltpu.sync_copy(zbuf, acc_sh)
        else:
            pltpu.sync_copy(zbuf, acc_sh.at[pl.ds(s * zl, zl)])
        plsc.subcore_barrier()

        def blk(i, _):
            base = wid * ew + i * bs
            pltpu.sync_copy(dst_hbm.at[pl.ds(base, bs)], dv)
            for b in range(4):
                pltpu.sync_copy(vals_hbm.at[b, pl.ds(base, bs)], vv)

                def vec(j, _):
                    iv[pl.ds(j * 16, 16)] = dv[pl.ds(j * 16, 16)] + (b * t)
                    return 0

                lax.fori_loop(0, bs // 16, vec, 0)
                pltpu.sync_copy(vv, acc_sh.at[iv], add=True)
            return 0

        lax.fori_loop(0, nb, blk, 0)
        plsc.subcore_barrier()
        if small:
            @pl.when(s == 0)
            def _():
                pltpu.sync_copy(acc_sh, out_hbm.at[pl.ds(c * 4 * t, 4 * t)])
        else:
            pltpu.sync_copy(acc_sh.at[pl.ds(s * zl, zl)],
                            out_hbm.at[pl.ds(c * 4 * t + s * zl, zl)])

    return k


@functools.lru_cache(maxsize=None)
def _make_sc_gatex(ep, t):
    ew, bs, nb = _blocks(ep)

    @functools.partial(
        pl.kernel,
        out_type=jax.ShapeDtypeStruct((4, ep), jnp.float32),
        mesh=_get_mesh(),
        compiler_params=_SC_PARAMS,
        scratch_types=[
            pltpu.VMEM((4 * t,), jnp.float32),
            pltpu.VMEM((4 * t,), jnp.float32),
            pltpu.VMEM((64,), jnp.float32),
            pltpu.VMEM((bs,), jnp.int32),
            pltpu.VMEM((bs,), jnp.int32),
            pltpu.VMEM((bs,), jnp.float32),
        ],
    )
    def k(hs_hbm, hd_hbm, c_hbm, src_hbm, dst_hbm, out_hbm,
          ts, td, tc, sv, dv, ov):
        wid = _wid()
        pltpu.sync_copy(hs_hbm, ts)
        pltpu.sync_copy(hd_hbm, td)
        pltpu.sync_copy(c_hbm, tc)

        def blk(i, _):
            base = wid * ew + i * bs
            pltpu.sync_copy(src_hbm.at[pl.ds(base, bs)], sv)
            pltpu.sync_copy(dst_hbm.at[pl.ds(base, bs)], dv)
            for b in range(4):
                cb = tc[pl.ds(b * 16, 16)]

                def vec(j, _):
                    sl = pl.ds(j * 16, 16)
                    a = plsc.load_gather(ts, [sv[sl] + (b * t)])
                    d = plsc.load_gather(td, [dv[sl] + (b * t)])
                    e = a + d
                    e = jnp.where(e >= 0.0, e, 0.2 * e)
                    ov[sl] = jnp.exp(e - cb)
                    return 0

                lax.fori_loop(0, bs // 16, vec, 0)
                pltpu.sync_copy(ov, out_hbm.at[b, pl.ds(base, bs)])
            return 0

        lax.fori_loop(0, nb, blk, 0)

    return k


@functools.lru_cache(maxsize=None)
def _make_sc_rows(ep, n, mp, scaled):
    ew, bs, nb = _blocks(ep)
    rpt = mp // NSUB

    scratch = [
        pltpu.VMEM((bs,), jnp.int32),
        pltpu.VMEM((bs,), jnp.int32),
        pltpu.VMEM((bs,), jnp.int32),
        pltpu.VMEM((bs,), jnp.int32),
        pltpu.VMEM((bs,), jnp.float32),
        pltpu.VMEM((bs, 128), jnp.float32),
        pltpu.VMEM((bs, 128), jnp.float32),
        pltpu.VMEM_SHARED((mp, 128), jnp.float32),
        pltpu.SemaphoreType.DMA,
        pltpu.SemaphoreType.DMA,
        pltpu.SemaphoreType.DMA,
        pltpu.SemaphoreType.DMA,
    ]

    def body(tabs, src_hbm, dst_hbm, w_hbm, out_hbm,
             sv0, sv1, dv0, dv1, wv, r0, r1, acc_sh,
             gsem0, gsem1, ssem0, ssem1):
        c = lax.axis_index("c")
        s = lax.axis_index("s")
        wid = _wid()
        svs, dvs, rs = (sv0, sv1), (dv0, dv1), (r0, r1)
        gsems, ssems = (gsem0, gsem1), (ssem0, ssem1)
        zb = min(bs, 128)
        nf, zt = rpt // zb, rpt % zb
        nco = (rpt + 127) // 128

        for bnum in range(4):
            tab = tabs[bnum]
            _zero_vmem2d(r0, zb)
            for tzi in range(nf):
                pltpu.sync_copy(r0.at[pl.ds(0, zb)],
                                acc_sh.at[pl.ds(s * rpt + tzi * zb, zb)])
            if zt:
                pltpu.sync_copy(r0.at[pl.ds(0, zt)],
                                acc_sh.at[pl.ds(s * rpt + nf * zb, zt)])
            plsc.subcore_barrier()

            pltpu.sync_copy(src_hbm.at[pl.ds(wid * ew, bs)], sv0)
            gd = [pltpu.async_copy(tab.at[sv0], r0, gsem0), None]
            sd = [None, None]
            for i in range(nb):
                cur = i & 1
                nxt = 1 - cur
                if i + 1 < nb:
                    nbase = wid * ew + (i + 1) * bs
                    pltpu.sync_copy(src_hbm.at[pl.ds(nbase, bs)], svs[nxt])
                    if sd[nxt] is not None:
                        sd[nxt].wait()
                        sd[nxt] = None
                    gd[nxt] = pltpu.async_copy(tab.at[svs[nxt]],
                                               rs[nxt], gsems[nxt])
                base = wid * ew + i * bs
                pltpu.sync_copy(dst_hbm.at[pl.ds(base, bs)], dvs[cur])
                if scaled:
                    pltpu.sync_copy(w_hbm.at[bnum, pl.ds(base, bs)], wv)
                gd[cur].wait()
                if scaled:
                    def scale(r, _, _rs=rs[cur]):
                        wr = plsc.load_gather(
                            wv, [jnp.full((16,), r, jnp.int32)])
                        for j in range(8):
                            sl = pl.ds(j * 16, 16)
                            _rs[r, sl] = _rs[r, sl] * wr
                        return 0
                    lax.fori_loop(0, bs, scale, 0)
                sd[cur] = pltpu.async_copy(rs[cur], acc_sh.at[dvs[cur]],
                                           ssems[cur], add=True)
            for d in sd:
                if d is not None:
                    d.wait()
            plsc.subcore_barrier()

            base_o = (2 * bnum + c) * mp + s * rpt
            for t2 in range(nco):
                sz = min(128, rpt - t2 * 128)
                pltpu.sync_copy(acc_sh.at[pl.ds(s * rpt + t2 * 128, sz)],
                                out_hbm.at[pl.ds(base_o + t2 * 128, sz)])

    if scaled:
        def body_s(t0, t1, t2, t3, src_hbm, dst_hbm, w_hbm, out_hbm, *scr):
            body((t0, t1, t2, t3), src_hbm, dst_hbm, w_hbm, out_hbm, *scr)
        fn = body_s
    else:
        def body_n(t0, t1, t2, t3, src_hbm, dst_hbm, out_hbm, *scr):
            body((t0, t1, t2, t3), src_hbm, dst_hbm, None, out_hbm, *scr)
        fn = body_n

    return pl.kernel(
        fn,
        out_type=jax.ShapeDtypeStruct((4 * 2 * mp, 128), jnp.float32),
        mesh=_get_mesh(),
        compiler_params=_SC_PARAMS,
        scratch_types=scratch,
    )


def _tc_dinv(degp, xp):
    def body(d_ref, x_ref, o_ref, xs_ref):
        d = d_ref[0] + d_ref[1] + 1.0
        di = lax.rsqrt(d.reshape(80, 128))
        o_ref[...] = di
        dflat = di.reshape(1, NP1)
        xs_ref[...] = (x_ref[...].reshape(BN, NP1)
                       * dflat).reshape(1, BN * NP1)

    dinv, xs = pl.pallas_call(
        body,
        out_shape=[jax.ShapeDtypeStruct((80, 128), jnp.float32),
                   jax.ShapeDtypeStruct((1, BN * NP1), jnp.float32)],
    )(degp.reshape(2, 80, 128), xp.reshape(1, BN * NP1))
    return dinv.reshape(NP1), xs.reshape(BN * NP1)


def _tc_gcn1ln(alphap, x, dinv, w1, g, b):
    blk = 512
    nj = NP1 // blk

    def body(a_ref, x_ref, di_ref, w_ref, g_ref, b_ref, o_ref):
        di = di_ref[0]
        alpha = di * (a_ref[0] + a_ref[1] + di * x_ref[0])
        w = w_ref[0]
        mw = jnp.mean(w)
        vw = jnp.mean((w - mw) ** 2)
        wc = w - mw
        denom = lax.rsqrt(alpha * alpha * vw + EPS)
        o_ref[0] = ((alpha * denom)[:, None] * wc[None, :] * g_ref[0][None, :]
                    + b_ref[0][None, :])

    grid = (BN, nj)
    return pl.pallas_call(
        body,
        grid=grid,
        in_specs=[
            pl.BlockSpec((2, blk), lambda b2, j: (0, b2 * nj + j)),
            pl.BlockSpec((1, blk), lambda b2, j: (0, b2 * nj + j)),
            pl.BlockSpec((1, blk), lambda b2, j: (0, j)),
            pl.BlockSpec((1, HID), lambda b2, j: (0, 0)),
            pl.BlockSpec((1, HID), lambda b2, j: (0, 0)),
            pl.BlockSpec((1, HID), lambda b2, j: (0, 0)),
        ],
        out_specs=pl.BlockSpec((1, blk, HID), lambda b2, j: (b2, j, 0)),
        out_shape=jax.ShapeDtypeStruct((BN, NP1, HID), jnp.float32),
    )(alphap.reshape(2, BN * NP1), x.reshape(1, BN * NP1),
      dinv.reshape(1, NP1), w1.reshape(1, HID),
      g.reshape(1, HID), b.reshape(1, HID))


def _tc_matmul(h, w):
    m, kdim = h.shape
    kout = w.shape[1]
    blk = 512

    def body(h_ref, w_ref, o_ref):
        o_ref[...] = jnp.dot(h_ref[...], w_ref[...],
                             preferred_element_type=jnp.float32)

    return pl.pallas_call(
        body,
        grid=(m // blk,),
        in_specs=[
            pl.BlockSpec((blk, kdim), lambda i: (i, 0)),
            pl.BlockSpec((kdim, kout), lambda i: (0, 0)),
        ],
        out_specs=pl.BlockSpec((blk, kout), lambda i: (i, 0)),
        out_shape=jax.ShapeDtypeStruct((m, kout), jnp.float32),
    )(h, w)


def _tc_matmul_scale(h, w, sc):
    m, kdim = h.shape
    kout = w.shape[1]
    blk = 512

    def body(h_ref, w_ref, s_ref, o_ref):
        o_ref[...] = jnp.dot(h_ref[...], w_ref[...],
                             preferred_element_type=jnp.float32) \
            * s_ref[0][:, None]

    return pl.pallas_call(
        body,
        grid=(m // blk,),
        in_specs=[
            pl.BlockSpec((blk, kdim), lambda i: (i, 0)),
            pl.BlockSpec((kdim, kout), lambda i: (0, 0)),
            pl.BlockSpec((1, blk), lambda i: (0, i)),
        ],
        out_specs=pl.BlockSpec((blk, kout), lambda i: (i, 0)),
        out_shape=jax.ShapeDtypeStruct((m, kout), jnp.float32),
    )(h, w, sc.reshape(1, m))


def _tc_scalerows(h, ex):
    bdim, ndim = h.shape[0], h.shape[1]
    blk = 512
    nj = ndim // blk

    def body(h_ref, e_ref, o_ref):
        o_ref[0] = h_ref[0] * e_ref[0][:, None]

    return pl.pallas_call(
        body,
        grid=(bdim, nj),
        in_specs=[
            pl.BlockSpec((1, blk, HID), lambda b, j: (b, j, 0)),
            pl.BlockSpec((1, blk), lambda b, j: (0, b * nj + j)),
        ],
        out_specs=pl.BlockSpec((1, blk, HID), lambda b, j: (b, j, 0)),
        out_shape=jax.ShapeDtypeStruct((bdim, ndim, HID), jnp.float32),
    )(h, ex.reshape(1, bdim * ndim))


def _tc_gcn2post(aggp, h2, h1, dinv, b2, g, b):
    blk = 512

    def body(a_ref, h2_ref, h1_ref, di_ref, b2_ref, g_ref, b_ref, o_ref):
        pre = (di_ref[0][:, None] * (a_ref[0, 0] + a_ref[0, 1] + h2_ref[0])
               + b2_ref[0][None, :])
        m = jnp.mean(pre, axis=-1, keepdims=True)
        v = jnp.mean((pre - m) ** 2, axis=-1, keepdims=True)
        o_ref[0] = h1_ref[0] + ((pre - m) * lax.rsqrt(v + EPS)
                                * g_ref[0][None, :] + b_ref[0][None, :])

    grid = (BN, NP1 // blk)
    return pl.pallas_call(
        body,
        grid=grid,
        in_specs=[
            pl.BlockSpec((1, 2, blk, HID), lambda b3, j: (b3, 0, j, 0)),
            pl.BlockSpec((1, blk, HID), lambda b3, j: (b3, j, 0)),
            pl.BlockSpec((1, blk, HID), lambda b3, j: (b3, j, 0)),
            pl.BlockSpec((1, blk), lambda b3, j: (0, j)),
            pl.BlockSpec((1, HID), lambda b3, j: (0, 0)),
            pl.BlockSpec((1, HID), lambda b3, j: (0, 0)),
            pl.BlockSpec((1, HID), lambda b3, j: (0, 0)),
        ],
        out_specs=pl.BlockSpec((1, blk, HID), lambda b3, j: (b3, j, 0)),
        out_shape=jax.ShapeDtypeStruct((BN, NP1, HID), jnp.float32),
    )(aggp, h2, h1, dinv.reshape(1, NP1), b2.reshape(1, HID),
      g.reshape(1, HID), b.reshape(1, HID))


def _tc_s2t_scores(hflat, p):
    m = hflat.shape[0]
    blk = 512
    f1w, f1b = p['fc1_W'], p['fc1_b']
    lng, lnb = p['ln_g'], p['ln_b']
    f2w = p['fc2_W'][:, 0]
    f2b = p['fc2_b'][0]

    def body(h_ref, w1_ref, b1_ref, g_ref, b_ref, w2_ref, b2_ref, o_ref):
        z = jnp.dot(h_ref[...], w1_ref[...],
                    preferred_element_type=jnp.float32) + b1_ref[0][None, :]
        mu = jnp.mean(z, axis=-1, keepdims=True)
        v = jnp.mean((z - mu) ** 2, axis=-1, keepdims=True)
        t = jnp.tanh((z - mu) * lax.rsqrt(v + EPS) * g_ref[0][None, :]
                     + b_ref[0][None, :])
        o_ref[0] = jnp.sum(t * w2_ref[0][None, :], axis=-1) + b2_ref[0, 0]

    return pl.pallas_call(
        body,
        grid=(m // blk,),
        in_specs=[
            pl.BlockSpec((blk, HID), lambda i: (i, 0)),
            pl.BlockSpec((HID, 4 * HID), lambda i: (0, 0)),
            pl.BlockSpec((1, 4 * HID), lambda i: (0, 0)),
            pl.BlockSpec((1, 4 * HID), lambda i: (0, 0)),
            pl.BlockSpec((1, 4 * HID), lambda i: (0, 0)),
            pl.BlockSpec((1, 4 * HID), lambda i: (0, 0)),
            pl.BlockSpec((1, 1), lambda i: (0, 0)),
        ],
        out_specs=pl.BlockSpec((1, blk), lambda i: (0, i)),
        out_shape=jax.ShapeDtypeStruct((1, m), jnp.float32),
    )(hflat, f1w, f1b.reshape(1, -1), lng.reshape(1, -1),
      lnb.reshape(1, -1), f2w.reshape(1, -1),
      p['fc2_b'].reshape(1, 1)).reshape(m)


def _tc_sexp(s, npad, nvalid):
    def body(s_ref, o_ref):
        row = s_ref[0]
        iot = lax.broadcasted_iota(jnp.int32, (npad,), 0)
        msk = iot < nvalid
        sm = jnp.where(msk, row, -1e30)
        cmax = jnp.max(sm)
        o_ref[0] = jnp.where(msk, jnp.exp(row - cmax), 0.0)

    return pl.pallas_call(
        body,
        grid=(BN,),
        in_specs=[pl.BlockSpec((1, npad), lambda b: (0, b))],
        out_specs=pl.BlockSpec((1, npad), lambda b: (0, b)),
        out_shape=jax.ShapeDtypeStruct((1, BN * npad), jnp.float32),
    )(s.reshape(1, BN * npad)).reshape(BN, npad)


def _tc_gatpre(lv2p, d0, d1, gw, a_s, a_d, nvalid):
    def body(l_ref, d0_ref, d1_ref, w_ref, as_ref, ad_ref, hw_ref, hs_ref,
             hd_ref, c_ref, es_ref):
        den = d0_ref[0] + d1_ref[0] + 1e-16
        lv2 = (l_ref[0, 0] + l_ref[0, 1]) / den[:, None]
        hw = jnp.dot(lv2, w_ref[...], preferred_element_type=jnp.float32)
        hs = jnp.sum(hw * as_ref[0][None, :], axis=-1)
        hd = jnp.sum(hw * ad_ref[0][None, :], axis=-1)
        iot = lax.broadcasted_iota(jnp.int32, (NP2,), 0)
        msk = iot < nvalid
        hs = jnp.where(msk, hs, 0.0)
        hd = jnp.where(msk, hd, 0.0)
        cb = jnp.maximum(jnp.max(jnp.where(msk, hs, -1e30))
                         + jnp.max(jnp.where(msk, hd, -1e30)), 0.0)
        e = hs + hd
        e = jnp.where(e >= 0.0, e, 0.2 * e)
        hw_ref[0] = hw
        hs_ref[0] = hs
        hd_ref[0] = hd
        c_ref[0] = jnp.full((NP2,), cb, jnp.float32)
        es_ref[0] = jnp.where(msk, jnp.exp(e - cb), 0.0)

    outs = pl.pallas_call(
        body,
        grid=(BN,),
        in_specs=[
            pl.BlockSpec((1, 2, NP2, HID), lambda b: (b, 0, 0, 0)),
            pl.BlockSpec((1, NP2), lambda b: (0, b)),
            pl.BlockSpec((1, NP2), lambda b: (0, b)),
            pl.BlockSpec((HID, HID), lambda b: (0, 0)),
            pl.BlockSpec((1, HID), lambda b: (0, 0)),
            pl.BlockSpec((1, HID), lambda b: (0, 0)),
        ],
        out_specs=[
            pl.BlockSpec((1, NP2, HID), lambda b: (b, 0, 0)),
            pl.BlockSpec((1, NP2), lambda b: (0, b)),
            pl.BlockSpec((1, NP2), lambda b: (0, b)),
            pl.BlockSpec((1, NP2), lambda b: (0, b)),
            pl.BlockSpec((1, NP2), lambda b: (0, b)),
        ],
        out_shape=[
            jax.ShapeDtypeStruct((BN, NP2, HID), jnp.float32),
            jax.ShapeDtypeStruct((1, BN * NP2), jnp.float32),
            jax.ShapeDtypeStruct((1, BN * NP2), jnp.float32),
            jax.ShapeDtypeStruct((1, BN * NP2), jnp.float32),
            jax.ShapeDtypeStruct((1, BN * NP2), jnp.float32),
        ],
    )(lv2p, d0.reshape(1, BN * NP2), d1.reshape(1, BN * NP2), gw,
      a_s.reshape(1, HID), a_d.reshape(1, HID))
    return outs


def _tc_gatpost(gaggp, hw, ex_self, d0, d1, gb, g, b):
    def body(a_ref, hw_ref, es_ref, d0_ref, d1_ref, gb_ref, g_ref, b_ref,
             o_ref):
        es = es_ref[0]
        den = d0_ref[0] + d1_ref[0] + es + 1e-16
        pre = ((a_ref[0, 0] + a_ref[0, 1] + es[:, None] * hw_ref[0])
               / den[:, None] + gb_ref[0][None, :])
        m = jnp.mean(pre, axis=-1, keepdims=True)
        v = jnp.mean((pre - m) ** 2, axis=-1, keepdims=True)
        o_ref[0] = ((pre - m) * lax.rsqrt(v + EPS) * g_ref[0][None, :]
                    + b_ref[0][None, :])

    return pl.pallas_call(
        body,
        grid=(BN,),
        in_specs=[
            pl.BlockSpec((1, 2, NP2, HID), lambda b2: (b2, 0, 0, 0)),
            pl.BlockSpec((1, NP2, HID), lambda b2: (b2, 0, 0)),
            pl.BlockSpec((1, NP2), lambda b2: (0, b2)),
            pl.BlockSpec((1, NP2), lambda b2: (0, b2)),
            pl.BlockSpec((1, NP2), lambda b2: (0, b2)),
            pl.BlockSpec((1, HID), lambda b2: (0, 0)),
            pl.BlockSpec((1, HID), lambda b2: (0, 0)),
            pl.BlockSpec((1, HID), lambda b2: (0, 0)),
        ],
        out_specs=pl.BlockSpec((1, NP2, HID), lambda b2: (b2, 0, 0)),
        out_shape=jax.ShapeDtypeStruct((BN, NP2, HID), jnp.float32),
    )(gaggp, hw, ex_self.reshape(1, BN * NP2), d0.reshape(1, BN * NP2),
      d1.reshape(1, BN * NP2), gb.reshape(1, HID),
      g.reshape(1, HID), b.reshape(1, HID))


def _tc_final(parts, d0, d1):
    def body(p_ref, d0_ref, d1_ref, o_ref):
        den = d0_ref[0] + d1_ref[0] + 1e-16
        o_ref[0] = (p_ref[0, 0] + p_ref[0, 1]) / den[:, None]

    return pl.pallas_call(
        body,
        grid=(BN,),
        in_specs=[
            pl.BlockSpec((1, 2, NP3, HID), lambda b: (b, 0, 0, 0)),
            pl.BlockSpec((1, NP3), lambda b: (0, b)),
            pl.BlockSpec((1, NP3), lambda b: (0, b)),
        ],
        out_specs=pl.BlockSpec((1, NP3, HID), lambda b: (b, 0, 0)),
        out_shape=jax.ShapeDtypeStruct((BN, NP3, HID), jnp.float32),
    )(parts, d0.reshape(1, BN * NP3), d1.reshape(1, BN * NP3))


def kernel(x, batch_num, level_11_edge_index, level_22_edge_index,
           level_21_seg, level_32_seg, level_13_indicator, params):
    p = params
    B = x.shape[0]

    i32 = jnp.int32
    src1 = level_11_edge_index[0].astype(i32)
    dst1 = level_11_edge_index[1].astype(i32)
    pad1 = jnp.full((EP1 - L1E,), NP1 - 1, i32)
    src1p = jnp.concatenate([src1, pad1])
    dst1p = jnp.concatenate([dst1, pad1])

    src2 = level_22_edge_index[0].astype(i32)
    dst2 = level_22_edge_index[1].astype(i32)
    pad2 = jnp.full((EP2 - L2E,), NP2 - 1, i32)
    src2p = jnp.concatenate([src2, pad2])
    dst2p = jnp.concatenate([dst2, pad2])

    seg21 = level_21_seg.astype(i32)
    seg21p = jnp.concatenate([seg21, jnp.full((SP1 - L1N,), NP2 - 1, i32)])
    iota1p = jnp.minimum(jnp.arange(SP1, dtype=i32), NP1 - 1)

    seg32 = level_32_seg.astype(i32)
    seg32p = jnp.concatenate([seg32, jnp.full((NP2 - L2N,), NP3 - 1, i32)])
    iota2p = jnp.arange(NP2, dtype=i32)

    xp = jnp.pad(x, ((0, 0), (0, NP1 - L1N)))

    degp = _make_sc_deg(EP1, NP1)(dst1p)
    dinv, xs = _tc_dinv(degp, xp)

    alphap = _make_sc_gcn1(EP1, NP1, NP1)(xs, src1p, dst1p)
    h1 = _tc_gcn1ln(alphap.reshape(2, BN, NP1), xp, dinv,
                    p['gcn1_W'][0], p['ln1_g'], p['ln1_b'])

    h1f = h1.reshape(BN * NP1, HID)
    dinv4 = jnp.tile(dinv, BN)
    h2s = _tc_matmul_scale(h1f, p['gcn2_W'], dinv4).reshape(BN, NP1, HID)
    rows1 = _make_sc_rows(EP1, NP1, NP1, False)
    aggp = rows1(h2s[0], h2s[1], h2s[2], h2s[3],
                 src1p, dst1p).reshape(BN, 2, NP1, HID)
    h = _tc_gcn2post(aggp, h2s, h1, dinv, p['gcn2_b'],
                     p['ln2_g'], p['ln2_b'])

    s12 = _tc_s2t_scores(h.reshape(BN * NP1, HID),
                         p['s2t12']).reshape(BN, NP1)
    ex12 = _tc_sexp(s12, NP1, L1N)
    ex12p = jnp.pad(ex12, ((0, 0), (0, SP1 - NP1)))
    den12 = _make_sc_segden(SP1, NP2)(ex12p, seg21p)
    d12 = den12.reshape(2, 4 * NP2)
    hs12 = _tc_scalerows(h, ex12)
    rows12 = _make_sc_rows(SP1, NP1, NP2, False)
    lv2p = rows12(hs12[0], hs12[1], hs12[2], hs12[3],
                  iota1p, seg21p).reshape(BN, 2, NP2, HID)

    hw, hs, hd, cb, ex_self = _tc_gatpre(lv2p, d12[0], d12[1], p['gat_W'],
                                         p['gat_as'], p['gat_ad'], L2N)
    cb64 = cb.reshape(BN, NP2)[:, :16].reshape(64)
    exg = _make_sc_gatex(EP2, NP2)(hs.reshape(4 * NP2), hd.reshape(4 * NP2),
                                   cb64, src2p, dst2p)
    deng = _make_sc_segden(EP2, NP2)(exg, dst2p)
    dg = deng.reshape(2, 4 * NP2)
    rowsg = _make_sc_rows(EP2, NP2, NP2, True)
    gaggp = rowsg(hw[0], hw[1], hw[2], hw[3],
                  src2p, dst2p, exg).reshape(BN, 2, NP2, HID)
    dgb = dg.reshape(2, BN, NP2)
    lv2f = _tc_gatpost(gaggp, hw, ex_self, dgb[0], dgb[1],
                       p['gat_b'], p['lng_g'], p['lng_b'])

    s23 = _tc_s2t_scores(lv2f.reshape(BN * NP2, HID),
                         p['s2t23']).reshape(BN, NP2)
    ex23 = _tc_sexp(s23, NP2, L2N)
    den23 = _make_sc_segden(NP2, NP3)(ex23, seg32p)
    d23 = den23.reshape(2, 4 * NP3)
    hs23 = _tc_scalerows(lv2f, ex23)
    rows23 = _make_sc_rows(NP2, NP2, NP3, False)
    lv3p = rows23(hs23[0], hs23[1], hs23[2], hs23[3],
                  iota2p, seg32p).reshape(BN, 2, NP3, HID)
    lv3 = _tc_final(lv3p, d23[0], d23[1])
    return lv3[:, :L3N, :]
